# baseline (speedup 1.0000x reference)
import functools
import jax
import jax.numpy as jnp
from jax import lax
from jax.experimental import pallas as pl
from jax.experimental.pallas import tpu as pltpu
from jax.experimental.pallas import tpu_sc as plsc

NC, NS, L = 2, 16, 16
NW = NC * NS
CHB = 128
ZN = 256
SPMEM_WORDS = 2097151


def _pick_nb(d, n_out_pad, mode):
    acc_words = (n_out_pad + 8) * d
    budget = (SPMEM_WORDS - acc_words - 65536) // NS
    for nb in (8, 4, 2, 1):
        ch = nb * CHB
        words = 2 * (ch * d + (ch * d if mode == "conv" else ch) + 2 * ch) + ZN * d
        if words <= budget:
            return nb
    raise ValueError("accumulator too large for Spmem")


def _tc(fn, n_rows, out_dims, big, small, R=2048):
    grid = (pl.cdiv(n_rows, R),)
    nb_, ns_ = len(big), len(small)

    def body(*refs):
        vals = [r[...] for r in refs[:nb_ + ns_]]
        outs = fn(*vals)
        if not isinstance(outs, (tuple, list)):
            outs = (outs,)
        for r, o in zip(refs[nb_ + ns_:], outs):
            r[...] = o

    in_specs = ([pl.BlockSpec((R, a.shape[1]), lambda i: (i, 0)) for a in big]
                + [pl.BlockSpec(w.shape, lambda i: (0,) * w.ndim) for w in small])
    out_specs = [pl.BlockSpec((R, d), lambda i: (i, 0)) for d in out_dims]
    out_shape = [jax.ShapeDtypeStruct((n_rows, d), jnp.float32) for d in out_dims]
    res = pl.pallas_call(body, grid=grid, in_specs=in_specs,
                         out_specs=out_specs, out_shape=out_shape)(*big, *small)
    return res if len(out_dims) > 1 else res[0]


@functools.lru_cache(maxsize=None)
def _edge_kernel(E_pad, n_src, d, n_out_pad, edge_split, mode, NB):
    CH = NB * CHB
    EPT = E_pad // (NW if edge_split else NS)
    nch = EPT // CH
    rpt = n_out_pad // NS
    n_acc = n_out_pad + 8
    E128 = E_pad // CHB
    mesh = plsc.VectorSubcoreMesh(core_axis_name="c", subcore_axis_name="s",
                                  num_cores=NC, num_subcores=NS)
    aux_scr = (pltpu.VMEM((2, CH, d), jnp.float32) if mode == "conv"
               else pltpu.VMEM((2, CH), jnp.float32))

    def body(y_hbm, s128, d128, aux_hbm, out_hbm, acc, sidx, didx, rows,
             aux_v, zbuf, sem_i, sem_g, sem_s):
        c = lax.axis_index("c")
        s = lax.axis_index("s")
        zvec = jnp.zeros((L,), jnp.float32)

        def zb(r, _):
            for j in range(d // L):
                zbuf[r, pl.ds(j * L, L)] = zvec
            return 0
        lax.fori_loop(0, ZN, zb, 0)

        base_r = s * rpt
        nzf, nzr = rpt // ZN, rpt % ZN

        def zc(r, _):
            pltpu.sync_copy(zbuf, acc.at[pl.ds(base_r + r * ZN, ZN)])
            return 0
        lax.fori_loop(0, nzf, zc, 0)
        if nzr:
            pltpu.sync_copy(zbuf.at[pl.ds(0, nzr)],
                            acc.at[pl.ds(base_r + nzf * ZN, nzr)])

        @pl.when(s == 0)
        def _():
            pltpu.sync_copy(zbuf.at[pl.ds(0, 8)], acc.at[pl.ds(n_out_pad, 8)])

        plsc.subcore_barrier()

        if edge_split:
            base128 = (c * NS + s) * (EPT // CHB)
        else:
            base128 = s * (EPT // CHB)

        def scatter_ops(b):
            ops = [(rows.at[b, pl.ds(j * CHB, CHB)], acc.at[didx.at[b, j]])
                   for j in range(NB)]
            if mode == "conv":
                ops += [(aux_v.at[b, pl.ds(j * CHB, CHB)], acc.at[didx.at[b, j]])
                        for j in range(NB)]
            return ops

        def phase(i, b):
            @pl.when(i >= 1)
            def _():
                for src, dst in scatter_ops(1 - b):
                    pltpu.make_async_copy(src, dst, sem_s).wait()
            @pl.when(i >= 1)
            def _():
                pltpu.make_async_copy(s128.at[pl.ds(0, NB)], sidx.at[b], sem_i).wait()
                pltpu.make_async_copy(s128.at[pl.ds(0, NB)], didx.at[b], sem_i).wait()
            @pl.when(i + 1 < nch)
            def _():
                offn = base128 + (i + 1) * NB
                pltpu.async_copy(s128.at[pl.ds(offn, NB)], sidx.at[1 - b], sem_i)
                pltpu.async_copy(d128.at[pl.ds(c * E128 + offn, NB)],
                                 didx.at[1 - b], sem_i)
            off128 = base128 + i * NB
            gs = [pltpu.async_copy(y_hbm.at[sidx.at[b, j]],
                                   rows.at[b, pl.ds(j * CHB, CHB)], sem_g)
                  for j in range(NB)]
            ax = pltpu.async_copy(aux_hbm.at[pl.ds(off128 * CHB, CH)],
                                  aux_v.at[b], sem_g)
            for g in gs:
                g.wait()
            ax.wait()
            if mode == "pool":
                def pm(e, _):
                    b16 = plsc.load_gather(aux_v.at[b],
                                           [jnp.full((L,), e, jnp.int32)])
                    for j in range(d // L):
                        rows[b, e, pl.ds(j * L, L)] = (
                            rows[b, e, pl.ds(j * L, L)] * b16)
                    return 0
                lax.fori_loop(0, CH, pm, 0)
            for src, dst in scatter_ops(b):
                pltpu.async_copy(src, dst, sem_s, add=True)

        def chunk2(i2, _):
            phase(2 * i2, 0)
            phase(2 * i2 + 1, 1)
            return 0
        pltpu.sync_copy(s128.at[pl.ds(base128, NB)], sidx.at[0])
        pltpu.sync_copy(d128.at[pl.ds(c * E128 + base128, NB)], didx.at[0])
        lax.fori_loop(0, nch // 2, chunk2, 0)
        for src, dst in scatter_ops((nch - 1) % 2):
            pltpu.make_async_copy(src, dst, sem_s).wait()

        plsc.subcore_barrier()

        def dc(r, _):
            pltpu.sync_copy(acc.at[pl.ds(base_r + r * ZN, ZN)],
                            out_hbm.at[pl.ds(c * n_out_pad + base_r + r * ZN, ZN)])
            return 0
        lax.fori_loop(0, nzf, dc, 0)
        if nzr:
            pltpu.sync_copy(acc.at[pl.ds(base_r + nzf * ZN, nzr)],
                            out_hbm.at[pl.ds(c * n_out_pad + base_r + nzf * ZN, nzr)])

    return pl.kernel(
        body,
        out_type=jax.ShapeDtypeStruct((NC * n_out_pad, d), jnp.float32),
        mesh=mesh,
        compiler_params=pltpu.CompilerParams(use_tc_tiling_on_sc=False,
                                             needs_layout_passes=False),
        scratch_types=[
            pltpu.VMEM_SHARED((n_acc, d), jnp.float32),
            pltpu.VMEM((2, NB, CHB), jnp.int32),
            pltpu.VMEM((2, NB, CHB), jnp.int32),
            pltpu.VMEM((2, CH, d), jnp.float32),
            aux_scr,
            pltpu.VMEM((ZN, d), jnp.float32),
            pltpu.SemaphoreType.DMA,
            pltpu.SemaphoreType.DMA,
            pltpu.SemaphoreType.DMA,
        ],
        name=f"edge_{mode}_{E_pad}_{n_src}_{d}_{n_out_pad}_{int(edge_split)}",
    )


def _ceil_to(x, m):
    return (x + m - 1) // m * m


def _pad_rows(a, n, val=0):
    if a.shape[0] == n:
        return a
    pad = [(0, n - a.shape[0])] + [(0, 0)] * (a.ndim - 1)
    return jnp.pad(a, pad, constant_values=val)


def _pad_cols(w, d):
    if w.shape[1] == d:
        return w
    return jnp.pad(w, [(0, 0), (0, d - w.shape[1])])


def _prep_edges(ei, n_dst_pad, E_pad, dst_split_half=None):
    src = _pad_rows(ei[0], E_pad, 0)
    if dst_split_half is None:
        dst = _pad_rows(ei[1], E_pad, n_dst_pad)
        d2 = jnp.stack([dst, dst])
    else:
        h = dst_split_half
        dst = _pad_rows(ei[1], E_pad, 2 * h)
        halves = []
        for c in range(2):
            dl = dst - c * h
            halves.append(jnp.where((dl >= 0) & (dl < h), dl, h))
        d2 = jnp.stack(halves)
    return src.reshape(E_pad // CHB, CHB), d2.reshape(2 * E_pad // CHB, CHB)


def _edge_pass(y, ei, aux, n_dst, mode, dst_split=False):
    E = ei.shape[1]
    d = y.shape[1]
    if dst_split:
        half = n_dst // 2
        n_out_pad = _ceil_to(half, NS * 8)
        nb = _pick_nb(d, n_out_pad, mode)
        E_pad = _ceil_to(E, NW * nb * CHB * 2)
        s128, d128 = _prep_edges(ei, None, E_pad, dst_split_half=half)
    else:
        n_out_pad = _ceil_to(n_dst, NS * 8)
        nb = _pick_nb(d, n_out_pad, mode)
        E_pad = _ceil_to(E, NW * nb * CHB * 2)
        s128, d128 = _prep_edges(ei, n_out_pad, E_pad)
    if mode == "conv":
        aux_p = _pad_rows(aux, E_pad, 0)
    else:
        aux_p = _pad_rows(aux.reshape(-1), E_pad, 0)
    k = _edge_kernel(E_pad, y.shape[0], d, n_out_pad, not dst_split, mode, nb)
    res = k(y, s128, d128, aux_p)
    if dst_split:
        half = n_dst // 2
        return jnp.concatenate([res[:half], res[n_out_pad:n_out_pad + half]], 0)
    return res[:n_dst], res[n_out_pad:n_out_pad + n_dst]


def _z_edges(ea, We, b, d):
    E = ea.shape[0]
    Wp = _pad_cols(We, d)
    bp = _pad_cols(b.reshape(1, -1), d)
    return _tc(lambda e_, w_, b_: jnp.dot(e_, w_, preferred_element_type=jnp.float32) + b_,
               E, [d], [ea], [Wp, bp], R=4096)


def kernel(xc, xf, ei_cf, ea_cf, ei_fp, ea_fp, ei_pp0, ea_pp0, ei_pp1, ea_pp1,
           ei_pp2, ea_pp2, ei_pp3, ea_pp3, ei_pc, ea_pc,
           ei_pool0, ea_pool0, ei_unpool0, ea_unpool0,
           ei_pool1, ea_pool1, ei_unpool1, ea_unpool1,
           ei_pool2, ea_pool2, ei_unpool2, ea_unpool2,
           W_cf, W_fp, Wm2, Ws2, Wm3, Ws3, Wm4a, Ws4a, Wm4b, Ws4b,
           Wm4c, Ws4c, Wm4d, Ws4d, Wm7, Ws7, Wm8, Ws8, Wm9, Ws9, W9b, Wf,
           b_cf, b_fp, bm2, bs2, bm3, bs3, bm4a, bs4a, bm4b, bs4b,
           bm4c, bs4c, bm4d, bs4d, bm7, bs7, bm8, bs8, bm9, bs9, b9b, bf):
    N_C, N_F = xc.shape[0], xf.shape[0]
    N_P0, N_P1, N_P2, N_P3 = 100000, 50000, 25000, 12500
    f32 = jnp.float32
    dot = lambda a, b: jnp.dot(a, b, preferred_element_type=f32)

    y_c = _tc(lambda x, w: dot(x, w), N_C, [16], [xc], [_pad_cols(W_cf[:2], 16)])
    z_cf = _z_edges(ea_cf, W_cf[2:6], b_cf, 16)
    aggF = _edge_pass(y_c, ei_cf, z_cf, N_F, "conv", dst_split=True)

    Wh = _pad_rows(W_fp[:12], 16)
    Wx = W_fp[12:16]
    y_f = _tc(lambda a, x, wh, wx: dot(jnp.maximum(a, 0.), wh) + dot(x, wx),
              N_F, [16], [aggF, xf], [Wh, Wx])
    z_fp = _z_edges(ea_fp, W_fp[16:20], b_fp, 16)
    q0, q1 = _edge_pass(y_f, ei_fp, z_fp, N_P0, "conv")
    c1 = _tc(lambda a, b: jnp.maximum(a + b, 0.), N_P0, [16], [q0, q1], [])

    q0, q1 = _edge_pass(c1, ei_pool0, ea_pool0, N_P1, "pool")
    y2, s2 = _tc(lambda a, b, wm, ws, bs: ((lambda p: (dot(p, wm), dot(p, ws) + bs))(a + b)),
                 N_P1, [16, 16], [q0, q1], [Wm2[:16], Ws2, bs2.reshape(1, -1)])
    z2 = _z_edges(ea_pp1, Wm2[16:20], bm2, 16)
    q0, q1 = _edge_pass(y2, ei_pp1, z2, N_P1, "conv")
    c2 = _tc(lambda a, b, s: jnp.maximum(a + b + s, 0.), N_P1, [16], [q0, q1, s2], [])

    q0, q1 = _edge_pass(c2, ei_pool1, ea_pool1, N_P2, "pool")
    y3, s3 = _tc(lambda a, b, wm, ws, bs: ((lambda p: (dot(p, wm), dot(p, ws) + bs))(a + b)),
                 N_P2, [16, 16], [q0, q1], [Wm3[:16], Ws3, bs3.reshape(1, -1)])
    z3 = _z_edges(ea_pp2, Wm3[16:20], bm3, 16)
    q0, q1 = _edge_pass(y3, ei_pp2, z3, N_P2, "conv")
    c3 = _tc(lambda a, b, s: jnp.maximum(a + b + s, 0.), N_P2, [16], [q0, q1, s3], [])

    q0, q1 = _edge_pass(c3, ei_pool2, ea_pool2, N_P3, "pool")
    p = _tc(lambda a, b: a + b, N_P3, [16], [q0, q1], [])
    for Wm, Ws, bm, bs in ((Wm4a, Ws4a, bm4a, bs4a), (Wm4b, Ws4b, bm4b, bs4b),
                           (Wm4c, Ws4c, bm4c, bs4c), (Wm4d, Ws4d, bm4d, bs4d)):
        dx = Wm.shape[0] - 4
        y4, s4 = _tc(lambda p_, wm, ws, bs_: (dot(p_, wm), dot(p_, ws) + bs_),
                     N_P3, [32, 32], [p], [Wm[:dx], Ws, bs.reshape(1, -1)])
        z4 = _z_edges(ea_pp3, Wm[dx:], bm, 32)
        q0, q1 = _edge_pass(y4, ei_pp3, z4, N_P3, "conv")
        p = _tc(lambda a, b, s: jnp.maximum(a + b + s, 0.), N_P3, [32], [q0, q1, s4], [])

    q0, q1 = _edge_pass(p, ei_unpool2, ea_unpool2, N_P2, "pool")
    y7, s7 = _tc(lambda a, b, c_, wm1, wm2_, ws1, ws2_, bs_:
                 ((lambda u: (dot(u, wm1) + dot(c_, wm2_),
                              dot(u, ws1) + dot(c_, ws2_) + bs_))(a + b)),
                 N_P2, [32, 32], [q0, q1, c3],
                 [Wm7[:32], Wm7[32:48], Ws7[:32], Ws7[32:48], bs7.reshape(1, -1)])
    z7 = _z_edges(ea_pp2, Wm7[48:52], bm7, 32)
    q0, q1 = _edge_pass(y7, ei_pp2, z7, N_P2, "conv")
    h7 = _tc(lambda a, b, s: jnp.maximum(a + b + s, 0.), N_P2, [32], [q0, q1, s7], [])

    q0, q1 = _edge_pass(h7, ei_unpool1, ea_unpool1, N_P1, "pool")
    y8, s8 = _tc(lambda a, b, c_, wm1, wm2_, ws1, ws2_, bs_:
                 ((lambda u: (dot(u, wm1) + dot(c_, wm2_),
                              dot(u, ws1) + dot(c_, ws2_) + bs_))(a + b)),
                 N_P1, [16, 16], [q0, q1, c2],
                 [Wm8[:32], Wm8[32:48], Ws8[:32], Ws8[32:48], bs8.reshape(1, -1)])
    z8 = _z_edges(ea_pp1, Wm8[48:52], bm8, 16)
    q0, q1 = _edge_pass(y8, ei_pp1, z8, N_P1, "conv")
    h8 = _tc(lambda a, b, s: jnp.maximum(a + b + s, 0.), N_P1, [16], [q0, q1, s8], [])

    q0, q1 = _edge_pass(h8, ei_unpool0, ea_unpool0, N_P0, "pool")
    y9, s9 = _tc(lambda a, b, c_, wm1, wm2_, ws1, ws2_, bs_:
                 ((lambda u: (dot(u, wm1) + dot(c_, wm2_),
                              dot(u, ws1) + dot(c_, ws2_) + bs_))(a + b)),
                 N_P0, [16, 16], [q0, q1, c1],
                 [Wm9[:16], Wm9[16:32], Ws9[:16], Ws9[16:32], bs9.reshape(1, -1)])
    z9 = _z_edges(ea_pp0, Wm9[32:36], bm9, 16)
    q0, q1 = _edge_pass(y9, ei_pp0, z9, N_P0, "conv")

    y9b = _tc(lambda a, b, s, w: dot(jnp.maximum(a + b + s, 0.), w),
              N_P0, [16], [q0, q1, s9], [W9b[:16]])
    z9b = _z_edges(ea_pc, W9b[16:20], b9b, 16)
    q0, q1 = _edge_pass(y9b, ei_pc, z9b, N_C, "conv")
    out = _tc(lambda a, b, wf, bf_: dot(jnp.maximum(a + b, 0.), wf) + bf_,
              N_C, [1], [q0, q1], [Wf, bf.reshape(1, -1)])
    return out


# speedup vs baseline: 1.1129x; 1.1129x over previous
import functools
import jax
import jax.numpy as jnp
from jax import lax
from jax.experimental import pallas as pl
from jax.experimental.pallas import tpu as pltpu
from jax.experimental.pallas import tpu_sc as plsc

NC, NS, L = 2, 16, 16
NW = NC * NS
CHB = 128
ZN = 256
SPMEM_WORDS = 2097151


def _pick_nb(d, n_out_pad, mode):
    acc_words = (n_out_pad + 8) * (d + (16 if mode == "conv2" else 0))
    budget = (SPMEM_WORDS - acc_words - 65536) // NS
    for nb in (8, 4, 2, 1):
        ch = nb * CHB
        aux_w = {"conv": ch * d, "conv2": ch * 16, "pool": ch}[mode]
        words = 2 * (ch * d + aux_w + 2 * ch) + ZN * d + (ZN * 16 if mode == "conv2" else 0)
        if words <= budget:
            return nb
    raise ValueError("accumulator too large for Spmem")


def _tc(fn, n_rows, out_dims, big, small, R=2048):
    grid = (pl.cdiv(n_rows, R),)
    nb_, ns_ = len(big), len(small)

    def body(*refs):
        vals = [r[...] for r in refs[:nb_ + ns_]]
        outs = fn(*vals)
        if not isinstance(outs, (tuple, list)):
            outs = (outs,)
        for r, o in zip(refs[nb_ + ns_:], outs):
            r[...] = o

    in_specs = ([pl.BlockSpec((R, a.shape[1]), lambda i: (i, 0)) for a in big]
                + [pl.BlockSpec(w.shape, lambda i: (0,) * w.ndim) for w in small])
    out_specs = [pl.BlockSpec((R, d), lambda i: (i, 0)) for d in out_dims]
    out_shape = [jax.ShapeDtypeStruct((n_rows, d), jnp.float32) for d in out_dims]
    res = pl.pallas_call(body, grid=grid, in_specs=in_specs,
                         out_specs=out_specs, out_shape=out_shape)(*big, *small)
    return res if len(out_dims) > 1 else res[0]


@functools.lru_cache(maxsize=None)
def _edge_kernel(E_pad, n_src, d, n_out_pad, edge_split, mode, NB):
    CH = NB * CHB
    EPT = E_pad // (NW if edge_split else NS)
    nch = EPT // CH
    rpt = n_out_pad // NS
    n_acc = n_out_pad + 8
    E128 = E_pad // CHB
    mesh = plsc.VectorSubcoreMesh(core_axis_name="c", subcore_axis_name="s",
                                  num_cores=NC, num_subcores=NS)
    aux_w = {"conv": d, "conv2": 16}.get(mode)
    aux_scr = (pltpu.VMEM((2, CH, aux_w), jnp.float32) if aux_w
               else pltpu.VMEM((2, CH), jnp.float32))
    conv2 = mode == "conv2"

    def body(*refs):
        if conv2:
            (y_hbm, s128, d128, aux_hbm, out_hbm, out2_hbm, acc, acc2,
             sidx, didx, rows, aux_v, zbuf, zbuf2, sem_i, sem_g, sem_s) = refs
        else:
            (y_hbm, s128, d128, aux_hbm, out_hbm, acc,
             sidx, didx, rows, aux_v, zbuf, sem_i, sem_g, sem_s) = refs
            out2_hbm = acc2 = zbuf2 = None
        c = lax.axis_index("c")
        s = lax.axis_index("s")
        zvec = jnp.zeros((L,), jnp.float32)

        def zb(r, _):
            for j in range(d // L):
                zbuf[r, pl.ds(j * L, L)] = zvec
            if conv2:
                zbuf2[r, :] = zvec
            return 0
        lax.fori_loop(0, ZN, zb, 0)

        base_r = s * rpt
        nzf, nzr = rpt // ZN, rpt % ZN

        def zero_drain(src_of, dst_of):
            def zc(r, _):
                for sr, dr in zip(src_of(base_r + r * ZN, ZN),
                                  dst_of(base_r + r * ZN, ZN)):
                    pltpu.sync_copy(sr, dr)
                return 0
            lax.fori_loop(0, nzf, zc, 0)
            if nzr:
                for sr, dr in zip(src_of(base_r + nzf * ZN, nzr),
                                  dst_of(base_r + nzf * ZN, nzr)):
                    pltpu.sync_copy(sr, dr)

        accs = [acc, acc2] if conv2 else [acc]
        zbufs = [zbuf, zbuf2] if conv2 else [zbuf]
        zero_drain(lambda o, n: [z.at[pl.ds(0, n)] for z in zbufs],
                   lambda o, n: [a.at[pl.ds(o, n)] for a in accs])

        @pl.when(s == 0)
        def _():
            for z, a in zip(zbufs, accs):
                pltpu.sync_copy(z.at[pl.ds(0, 8)], a.at[pl.ds(n_out_pad, 8)])

        plsc.subcore_barrier()

        if edge_split:
            base128 = (c * NS + s) * (EPT // CHB)
        else:
            base128 = s * (EPT // CHB)

        def scatter_ops(b):
            ops = [(rows.at[b, pl.ds(j * CHB, CHB)], acc.at[didx.at[b, j]])
                   for j in range(NB)]
            if mode == "conv":
                ops += [(aux_v.at[b, pl.ds(j * CHB, CHB)], acc.at[didx.at[b, j]])
                        for j in range(NB)]
            elif conv2:
                ops += [(aux_v.at[b, pl.ds(j * CHB, CHB)], acc2.at[didx.at[b, j]])
                        for j in range(NB)]
            return ops

        def phase(i, b):
            @pl.when(i >= 1)
            def _():
                pltpu.make_async_copy(s128.at[pl.ds(0, NB)], sidx.at[b], sem_i).wait()
                pltpu.make_async_copy(s128.at[pl.ds(0, NB)], didx.at[b], sem_i).wait()
            off128 = base128 + i * NB
            gs = [pltpu.async_copy(y_hbm.at[sidx.at[b, j]],
                                   rows.at[b, pl.ds(j * CHB, CHB)], sem_g)
                  for j in range(NB)]
            gs.append(pltpu.async_copy(aux_hbm.at[pl.ds(off128 * CHB, CH)],
                                       aux_v.at[b], sem_g))
            @pl.when(i >= 1)
            def _():
                for src, dst in scatter_ops(1 - b):
                    pltpu.make_async_copy(src, dst, sem_s).wait()
            @pl.when(i + 1 < nch)
            def _():
                offn = base128 + (i + 1) * NB
                pltpu.async_copy(s128.at[pl.ds(offn, NB)], sidx.at[1 - b], sem_i)
                pltpu.async_copy(d128.at[pl.ds(c * E128 + offn, NB)],
                                 didx.at[1 - b], sem_i)
            for g in gs:
                g.wait()
            if mode == "pool":
                def pm(e, _):
                    b16 = plsc.load_gather(aux_v.at[b],
                                           [jnp.full((L,), e, jnp.int32)])
                    for j in range(d // L):
                        rows[b, e, pl.ds(j * L, L)] = (
                            rows[b, e, pl.ds(j * L, L)] * b16)
                    return 0
                lax.fori_loop(0, CH, pm, 0)
            for src, dst in scatter_ops(b):
                pltpu.async_copy(src, dst, sem_s, add=True)

        def chunk2(i2, _):
            phase(2 * i2, 0)
            phase(2 * i2 + 1, 1)
            return 0
        pltpu.sync_copy(s128.at[pl.ds(base128, NB)], sidx.at[0])
        pltpu.sync_copy(d128.at[pl.ds(c * E128 + base128, NB)], didx.at[0])
        lax.fori_loop(0, nch // 2, chunk2, 0)
        for src, dst in scatter_ops((nch - 1) % 2):
            pltpu.make_async_copy(src, dst, sem_s).wait()

        plsc.subcore_barrier()

        outs = [out_hbm, out2_hbm] if conv2 else [out_hbm]
        zero_drain(lambda o, n: [a.at[pl.ds(o, n)] for a in accs],
                   lambda o, n: [h.at[pl.ds(c * n_out_pad + o, n)] for h in outs])

    out_type = jax.ShapeDtypeStruct((NC * n_out_pad, d), jnp.float32)
    if conv2:
        out_type = (out_type,
                    jax.ShapeDtypeStruct((NC * n_out_pad, 16), jnp.float32))
    scratch = [pltpu.VMEM_SHARED((n_acc, d), jnp.float32)]
    if conv2:
        scratch.append(pltpu.VMEM_SHARED((n_acc, 16), jnp.float32))
    scratch += [
        pltpu.VMEM((2, NB, CHB), jnp.int32),
        pltpu.VMEM((2, NB, CHB), jnp.int32),
        pltpu.VMEM((2, CH, d), jnp.float32),
        aux_scr,
        pltpu.VMEM((ZN, d), jnp.float32),
    ]
    if conv2:
        scratch.append(pltpu.VMEM((ZN, 16), jnp.float32))
    scratch += [pltpu.SemaphoreType.DMA] * 3
    return pl.kernel(
        body,
        out_type=out_type,
        mesh=mesh,
        compiler_params=pltpu.CompilerParams(use_tc_tiling_on_sc=False,
                                             needs_layout_passes=False),
        scratch_types=scratch,
        name=f"edge_{mode}_{E_pad}_{n_src}_{d}_{n_out_pad}_{int(edge_split)}",
    )


def _ceil_to(x, m):
    return (x + m - 1) // m * m


def _pad_rows(a, n, val=0):
    if a.shape[0] == n:
        return a
    pad = [(0, n - a.shape[0])] + [(0, 0)] * (a.ndim - 1)
    return jnp.pad(a, pad, constant_values=val)


def _pad_cols(w, d):
    if w.shape[1] == d:
        return w
    return jnp.pad(w, [(0, 0), (0, d - w.shape[1])])


def _prep_edges(ei, n_dst_pad, E_pad, dst_split_half=None):
    src = _pad_rows(ei[0], E_pad, 0)
    if dst_split_half is None:
        dst = _pad_rows(ei[1], E_pad, n_dst_pad)
        d2 = jnp.stack([dst, dst])
    else:
        h = dst_split_half
        dst = _pad_rows(ei[1], E_pad, 2 * h)
        halves = []
        for c in range(2):
            dl = dst - c * h
            halves.append(jnp.where((dl >= 0) & (dl < h), dl, h))
        d2 = jnp.stack(halves)
    return src.reshape(E_pad // CHB, CHB), d2.reshape(2 * E_pad // CHB, CHB)


def _edge_pass(y, ei, aux, n_dst, mode, dst_split=False):
    E = ei.shape[1]
    d = y.shape[1]
    if dst_split:
        half = n_dst // 2
        n_out_pad = _ceil_to(half, NS * 8)
        nb = _pick_nb(d, n_out_pad, mode)
        E_pad = _ceil_to(E, NW * nb * CHB * 2)
        s128, d128 = _prep_edges(ei, None, E_pad, dst_split_half=half)
    else:
        n_out_pad = _ceil_to(n_dst, NS * 8)
        nb = _pick_nb(d, n_out_pad, mode)
        E_pad = _ceil_to(E, NW * nb * CHB * 2)
        s128, d128 = _prep_edges(ei, n_out_pad, E_pad)
    if mode == "conv":
        aux_p = _pad_rows(aux, E_pad, 0)
    elif mode == "conv2":
        ea16 = jnp.concatenate(
            [aux, jnp.ones((E, 1), jnp.float32),
             jnp.zeros((E, 11), jnp.float32)], axis=1)
        aux_p = _pad_rows(ea16, E_pad, 0)
    else:
        aux_p = _pad_rows(aux.reshape(-1), E_pad, 0)
    k = _edge_kernel(E_pad, y.shape[0], d, n_out_pad, not dst_split, mode, nb)
    res = k(y, s128, d128, aux_p)
    if mode == "conv2":
        res, res2 = res
        return (res[:n_dst], res[n_out_pad:n_out_pad + n_dst],
                res2[:n_dst], res2[n_out_pad:n_out_pad + n_dst])
    if dst_split:
        half = n_dst // 2
        return jnp.concatenate([res[:half], res[n_out_pad:n_out_pad + half]], 0)
    return res[:n_dst], res[n_out_pad:n_out_pad + n_dst]


def _z_edges(ea, We, b, d):
    E = ea.shape[0]
    Wp = _pad_cols(We, d)
    bp = _pad_cols(b.reshape(1, -1), d)
    return _tc(lambda e_, w_, b_: jnp.dot(e_, w_, preferred_element_type=jnp.float32) + b_,
               E, [d], [ea], [Wp, bp], R=4096)


def _wz(We, b, d):
    return jnp.concatenate([_pad_cols(We, d), _pad_cols(b.reshape(1, -1), d),
                            jnp.zeros((11, d), jnp.float32)], axis=0)


def kernel(xc, xf, ei_cf, ea_cf, ei_fp, ea_fp, ei_pp0, ea_pp0, ei_pp1, ea_pp1,
           ei_pp2, ea_pp2, ei_pp3, ea_pp3, ei_pc, ea_pc,
           ei_pool0, ea_pool0, ei_unpool0, ea_unpool0,
           ei_pool1, ea_pool1, ei_unpool1, ea_unpool1,
           ei_pool2, ea_pool2, ei_unpool2, ea_unpool2,
           W_cf, W_fp, Wm2, Ws2, Wm3, Ws3, Wm4a, Ws4a, Wm4b, Ws4b,
           Wm4c, Ws4c, Wm4d, Ws4d, Wm7, Ws7, Wm8, Ws8, Wm9, Ws9, W9b, Wf,
           b_cf, b_fp, bm2, bs2, bm3, bs3, bm4a, bs4a, bm4b, bs4b,
           bm4c, bs4c, bm4d, bs4d, bm7, bs7, bm8, bs8, bm9, bs9, b9b, bf):
    N_C, N_F = xc.shape[0], xf.shape[0]
    N_P0, N_P1, N_P2, N_P3 = 100000, 50000, 25000, 12500
    f32 = jnp.float32
    dot = lambda a, b: jnp.dot(a, b, preferred_element_type=f32)

    y_c = _tc(lambda x, w: dot(x, w), N_C, [16], [xc], [_pad_cols(W_cf[:2], 16)])
    z_cf = _z_edges(ea_cf, W_cf[2:6], b_cf, 16)
    aggF = _edge_pass(y_c, ei_cf, z_cf, N_F, "conv", dst_split=True)

    Wh = _pad_rows(W_fp[:12], 16)
    Wx = W_fp[12:16]
    y_f = _tc(lambda a, x, wh, wx: dot(jnp.maximum(a, 0.), wh) + dot(x, wx),
              N_F, [16], [aggF, xf], [Wh, Wx])
    z_fp = _z_edges(ea_fp, W_fp[16:20], b_fp, 16)
    q0, q1 = _edge_pass(y_f, ei_fp, z_fp, N_P0, "conv")
    c1 = _tc(lambda a, b: jnp.maximum(a + b, 0.), N_P0, [16], [q0, q1], [])

    q0, q1 = _edge_pass(c1, ei_pool0, ea_pool0, N_P1, "pool")
    y2, s2 = _tc(lambda a, b, wm, ws, bs: ((lambda p: (dot(p, wm), dot(p, ws) + bs))(a + b)),
                 N_P1, [16, 16], [q0, q1], [Wm2[:16], Ws2, bs2.reshape(1, -1)])
    z2 = _z_edges(ea_pp1, Wm2[16:20], bm2, 16)
    q0, q1 = _edge_pass(y2, ei_pp1, z2, N_P1, "conv")
    c2 = _tc(lambda a, b, s: jnp.maximum(a + b + s, 0.), N_P1, [16], [q0, q1, s2], [])

    q0, q1 = _edge_pass(c2, ei_pool1, ea_pool1, N_P2, "pool")
    y3, s3 = _tc(lambda a, b, wm, ws, bs: ((lambda p: (dot(p, wm), dot(p, ws) + bs))(a + b)),
                 N_P2, [16, 16], [q0, q1], [Wm3[:16], Ws3, bs3.reshape(1, -1)])
    q0, q1, A0, A1 = _edge_pass(y3, ei_pp2, ea_pp2, N_P2, "conv2")
    c3 = _tc(lambda a, b, A, B, s, wz: jnp.maximum(a + b + dot(A + B, wz) + s, 0.),
             N_P2, [16], [q0, q1, A0, A1, s3], [_wz(Wm3[16:20], bm3, 16)])

    q0, q1 = _edge_pass(c3, ei_pool2, ea_pool2, N_P3, "pool")
    p = _tc(lambda a, b: a + b, N_P3, [16], [q0, q1], [])
    for Wm, Ws, bm, bs in ((Wm4a, Ws4a, bm4a, bs4a), (Wm4b, Ws4b, bm4b, bs4b),
                           (Wm4c, Ws4c, bm4c, bs4c), (Wm4d, Ws4d, bm4d, bs4d)):
        dx = Wm.shape[0] - 4
        y4, s4 = _tc(lambda p_, wm, ws, bs_: (dot(p_, wm), dot(p_, ws) + bs_),
                     N_P3, [32, 32], [p], [Wm[:dx], Ws, bs.reshape(1, -1)])
        q0, q1, A0, A1 = _edge_pass(y4, ei_pp3, ea_pp3, N_P3, "conv2")
        p = _tc(lambda a, b, A, B, s, wz: jnp.maximum(a + b + dot(A + B, wz) + s, 0.),
                N_P3, [32], [q0, q1, A0, A1, s4], [_wz(Wm[dx:], bm, 32)])

    q0, q1 = _edge_pass(p, ei_unpool2, ea_unpool2, N_P2, "pool")
    y7, s7 = _tc(lambda a, b, c_, wm1, wm2_, ws1, ws2_, bs_:
                 ((lambda u: (dot(u, wm1) + dot(c_, wm2_),
                              dot(u, ws1) + dot(c_, ws2_) + bs_))(a + b)),
                 N_P2, [32, 32], [q0, q1, c3],
                 [Wm7[:32], Wm7[32:48], Ws7[:32], Ws7[32:48], bs7.reshape(1, -1)])
    q0, q1, A0, A1 = _edge_pass(y7, ei_pp2, ea_pp2, N_P2, "conv2")
    h7 = _tc(lambda a, b, A, B, s, wz: jnp.maximum(a + b + dot(A + B, wz) + s, 0.),
             N_P2, [32], [q0, q1, A0, A1, s7], [_wz(Wm7[48:52], bm7, 32)])

    q0, q1 = _edge_pass(h7, ei_unpool1, ea_unpool1, N_P1, "pool")
    y8, s8 = _tc(lambda a, b, c_, wm1, wm2_, ws1, ws2_, bs_:
                 ((lambda u: (dot(u, wm1) + dot(c_, wm2_),
                              dot(u, ws1) + dot(c_, ws2_) + bs_))(a + b)),
                 N_P1, [16, 16], [q0, q1, c2],
                 [Wm8[:32], Wm8[32:48], Ws8[:32], Ws8[32:48], bs8.reshape(1, -1)])
    z8 = _z_edges(ea_pp1, Wm8[48:52], bm8, 16)
    q0, q1 = _edge_pass(y8, ei_pp1, z8, N_P1, "conv")
    h8 = _tc(lambda a, b, s: jnp.maximum(a + b + s, 0.), N_P1, [16], [q0, q1, s8], [])

    q0, q1 = _edge_pass(h8, ei_unpool0, ea_unpool0, N_P0, "pool")
    y9, s9 = _tc(lambda a, b, c_, wm1, wm2_, ws1, ws2_, bs_:
                 ((lambda u: (dot(u, wm1) + dot(c_, wm2_),
                              dot(u, ws1) + dot(c_, ws2_) + bs_))(a + b)),
                 N_P0, [16, 16], [q0, q1, c1],
                 [Wm9[:16], Wm9[16:32], Ws9[:16], Ws9[16:32], bs9.reshape(1, -1)])
    z9 = _z_edges(ea_pp0, Wm9[32:36], bm9, 16)
    q0, q1 = _edge_pass(y9, ei_pp0, z9, N_P0, "conv")

    y9b = _tc(lambda a, b, s, w: dot(jnp.maximum(a + b + s, 0.), w),
              N_P0, [16], [q0, q1, s9], [W9b[:16]])
    z9b = _z_edges(ea_pc, W9b[16:20], b9b, 16)
    q0, q1 = _edge_pass(y9b, ei_pc, z9b, N_C, "conv")
    out = _tc(lambda a, b, wf, bf_: dot(jnp.maximum(a + b, 0.), wf) + bf_,
              N_C, [1], [q0, q1], [Wf, bf.reshape(1, -1)])
    return out


# speedup vs baseline: 1.2600x; 1.1322x over previous
import functools
import jax
import jax.numpy as jnp
from jax import lax
from jax.experimental import pallas as pl
from jax.experimental.pallas import tpu as pltpu
from jax.experimental.pallas import tpu_sc as plsc

NC, NS, L = 2, 16, 16
NW = NC * NS
CHB = 128
ZN = 256
SPMEM_WORDS = 2097151


def _pick_nb(d, n_out_pad, mode):
    acc_words = (n_out_pad + 8) * (d + (16 if mode == "conv2" else 0))
    budget = (SPMEM_WORDS - acc_words - 65536) // NS
    for nb in (8, 4, 2, 1):
        ch = nb * CHB
        aux_w = {"conv": ch * d, "conv2": ch * 16, "pool": ch}[mode]
        words = 2 * (ch * d + aux_w + 2 * ch) + ZN * d + (ZN * 16 if mode == "conv2" else 0)
        if words <= budget:
            return nb
    raise ValueError("accumulator too large for Spmem")


def _tc(fn, n_rows, out_dims, big, small, R=2048):
    grid = (pl.cdiv(n_rows, R),)
    nb_, ns_ = len(big), len(small)

    def body(*refs):
        vals = [r[...] for r in refs[:nb_ + ns_]]
        outs = fn(*vals)
        if not isinstance(outs, (tuple, list)):
            outs = (outs,)
        for r, o in zip(refs[nb_ + ns_:], outs):
            r[...] = o

    in_specs = ([pl.BlockSpec((R, a.shape[1]), lambda i: (i, 0)) for a in big]
                + [pl.BlockSpec(w.shape, lambda i: (0,) * w.ndim) for w in small])
    out_specs = [pl.BlockSpec((R, d), lambda i: (i, 0)) for d in out_dims]
    out_shape = [jax.ShapeDtypeStruct((n_rows, d), jnp.float32) for d in out_dims]
    res = pl.pallas_call(body, grid=grid, in_specs=in_specs,
                         out_specs=out_specs, out_shape=out_shape)(*big, *small)
    return res if len(out_dims) > 1 else res[0]


@functools.lru_cache(maxsize=None)
def _edge_kernel(E_pad, n_src, d, n_out_pad, edge_split, mode, NB):
    CH = NB * CHB
    EPT = E_pad // (NW if edge_split else NS)
    nch = EPT // CH
    rpt = n_out_pad // NS
    n_acc = n_out_pad + 8
    E128 = E_pad // CHB
    mesh = plsc.VectorSubcoreMesh(core_axis_name="c", subcore_axis_name="s",
                                  num_cores=NC, num_subcores=NS)
    aux_w = {"conv": d, "conv2": 16}.get(mode)
    aux_scr = (pltpu.VMEM((2, CH, aux_w), jnp.float32) if aux_w
               else pltpu.VMEM((2, CH), jnp.float32))
    conv2 = mode == "conv2"

    def body(*refs):
        if conv2:
            (y_hbm, s128, d128, aux_hbm, out_hbm, out2_hbm, acc, acc2,
             sidx, didx, rows, aux_v, zbuf, zbuf2, sem_i, sem_g, sem_s) = refs
        else:
            (y_hbm, s128, d128, aux_hbm, out_hbm, acc,
             sidx, didx, rows, aux_v, zbuf, sem_i, sem_g, sem_s) = refs
            out2_hbm = acc2 = zbuf2 = None
        c = lax.axis_index("c")
        s = lax.axis_index("s")
        zvec = jnp.zeros((L,), jnp.float32)

        def zb(r, _):
            for j in range(d // L):
                zbuf[r, pl.ds(j * L, L)] = zvec
            if conv2:
                zbuf2[r, :] = zvec
            return 0
        lax.fori_loop(0, ZN, zb, 0)

        base_r = s * rpt
        nzf, nzr = rpt // ZN, rpt % ZN

        def zero_drain(src_of, dst_of):
            def zc(r, _):
                for sr, dr in zip(src_of(base_r + r * ZN, ZN),
                                  dst_of(base_r + r * ZN, ZN)):
                    pltpu.sync_copy(sr, dr)
                return 0
            lax.fori_loop(0, nzf, zc, 0)
            if nzr:
                for sr, dr in zip(src_of(base_r + nzf * ZN, nzr),
                                  dst_of(base_r + nzf * ZN, nzr)):
                    pltpu.sync_copy(sr, dr)

        accs = [acc, acc2] if conv2 else [acc]
        zbufs = [zbuf, zbuf2] if conv2 else [zbuf]
        zero_drain(lambda o, n: [z.at[pl.ds(0, n)] for z in zbufs],
                   lambda o, n: [a.at[pl.ds(o, n)] for a in accs])

        @pl.when(s == 0)
        def _():
            for z, a in zip(zbufs, accs):
                pltpu.sync_copy(z.at[pl.ds(0, 8)], a.at[pl.ds(n_out_pad, 8)])

        plsc.subcore_barrier()

        if edge_split:
            base128 = (c * NS + s) * (EPT // CHB)
        else:
            base128 = s * (EPT // CHB)

        def scatter_ops(b):
            ops = [(rows.at[b, pl.ds(j * CHB, CHB)], acc.at[didx.at[b, j]])
                   for j in range(NB)]
            if mode == "conv":
                ops += [(aux_v.at[b, pl.ds(j * CHB, CHB)], acc.at[didx.at[b, j]])
                        for j in range(NB)]
            elif conv2:
                ops += [(aux_v.at[b, pl.ds(j * CHB, CHB)], acc2.at[didx.at[b, j]])
                        for j in range(NB)]
            return ops

        def phase(i, b):
            @pl.when(i >= 1)
            def _():
                pltpu.make_async_copy(s128.at[pl.ds(0, NB)], sidx.at[b], sem_i).wait()
                pltpu.make_async_copy(s128.at[pl.ds(0, NB)], didx.at[b], sem_i).wait()
            off128 = base128 + i * NB
            gs = [pltpu.async_copy(y_hbm.at[sidx.at[b, j]],
                                   rows.at[b, pl.ds(j * CHB, CHB)], sem_g)
                  for j in range(NB)]
            gs.append(pltpu.async_copy(aux_hbm.at[pl.ds(off128 * CHB, CH)],
                                       aux_v.at[b], sem_g))
            @pl.when(i >= 1)
            def _():
                for src, dst in scatter_ops(1 - b):
                    pltpu.make_async_copy(src, dst, sem_s).wait()
            @pl.when(i + 1 < nch)
            def _():
                offn = base128 + (i + 1) * NB
                pltpu.async_copy(s128.at[pl.ds(offn, NB)], sidx.at[1 - b], sem_i)
                pltpu.async_copy(d128.at[pl.ds(c * E128 + offn, NB)],
                                 didx.at[1 - b], sem_i)
            for g in gs:
                g.wait()
            if mode == "pool":
                @plsc.parallel_loop(0, CH, 1, unroll=4)
                def pm(e):
                    b16 = plsc.load_gather(aux_v.at[b],
                                           [jnp.full((L,), e, jnp.int32)])
                    for j in range(d // L):
                        rows[b, e, pl.ds(j * L, L)] = (
                            rows[b, e, pl.ds(j * L, L)] * b16)
            for src, dst in scatter_ops(b):
                pltpu.async_copy(src, dst, sem_s, add=True)

        def chunk2(i2, _):
            phase(2 * i2, 0)
            phase(2 * i2 + 1, 1)
            return 0
        pltpu.sync_copy(s128.at[pl.ds(base128, NB)], sidx.at[0])
        pltpu.sync_copy(d128.at[pl.ds(c * E128 + base128, NB)], didx.at[0])
        lax.fori_loop(0, nch // 2, chunk2, 0)
        for src, dst in scatter_ops((nch - 1) % 2):
            pltpu.make_async_copy(src, dst, sem_s).wait()

        plsc.subcore_barrier()

        outs = [out_hbm, out2_hbm] if conv2 else [out_hbm]
        zero_drain(lambda o, n: [a.at[pl.ds(o, n)] for a in accs],
                   lambda o, n: [h.at[pl.ds(c * n_out_pad + o, n)] for h in outs])

    out_type = jax.ShapeDtypeStruct((NC * n_out_pad, d), jnp.float32)
    if conv2:
        out_type = (out_type,
                    jax.ShapeDtypeStruct((NC * n_out_pad, 16), jnp.float32))
    scratch = [pltpu.VMEM_SHARED((n_acc, d), jnp.float32)]
    if conv2:
        scratch.append(pltpu.VMEM_SHARED((n_acc, 16), jnp.float32))
    scratch += [
        pltpu.VMEM((2, NB, CHB), jnp.int32),
        pltpu.VMEM((2, NB, CHB), jnp.int32),
        pltpu.VMEM((2, CH, d), jnp.float32),
        aux_scr,
        pltpu.VMEM((ZN, d), jnp.float32),
    ]
    if conv2:
        scratch.append(pltpu.VMEM((ZN, 16), jnp.float32))
    scratch += [pltpu.SemaphoreType.DMA] * 3
    return pl.kernel(
        body,
        out_type=out_type,
        mesh=mesh,
        compiler_params=pltpu.CompilerParams(use_tc_tiling_on_sc=False,
                                             needs_layout_passes=False),
        scratch_types=scratch,
        name=f"edge_{mode}_{E_pad}_{n_src}_{d}_{n_out_pad}_{int(edge_split)}",
    )


def _ceil_to(x, m):
    return (x + m - 1) // m * m


def _pad_rows(a, n, val=0):
    if a.shape[0] == n:
        return a
    pad = [(0, n - a.shape[0])] + [(0, 0)] * (a.ndim - 1)
    return jnp.pad(a, pad, constant_values=val)


def _pad_cols(w, d):
    if w.shape[1] == d:
        return w
    return jnp.pad(w, [(0, 0), (0, d - w.shape[1])])


def _prep_edges(ei, n_dst_pad, E_pad, dst_split_half=None):
    src = _pad_rows(ei[0], E_pad, 0)
    if dst_split_half is None:
        dst = _pad_rows(ei[1], E_pad, n_dst_pad)
        d2 = jnp.stack([dst, dst])
    else:
        h = dst_split_half
        dst = _pad_rows(ei[1], E_pad, 2 * h)
        halves = []
        for c in range(2):
            dl = dst - c * h
            halves.append(jnp.where((dl >= 0) & (dl < h), dl, h))
        d2 = jnp.stack(halves)
    return src.reshape(E_pad // CHB, CHB), d2.reshape(2 * E_pad // CHB, CHB)


def _edge_pass(y, ei, aux, n_dst, mode, dst_split=False):
    E = ei.shape[1]
    d = y.shape[1]
    if dst_split:
        half = n_dst // 2
        n_out_pad = _ceil_to(half, NS * 8)
        nb = _pick_nb(d, n_out_pad, mode)
        E_pad = _ceil_to(E, NW * nb * CHB * 2)
        s128, d128 = _prep_edges(ei, None, E_pad, dst_split_half=half)
    else:
        n_out_pad = _ceil_to(n_dst, NS * 8)
        nb = _pick_nb(d, n_out_pad, mode)
        E_pad = _ceil_to(E, NW * nb * CHB * 2)
        s128, d128 = _prep_edges(ei, n_out_pad, E_pad)
    if mode == "conv":
        aux_p = aux(E_pad)
    elif mode == "conv2":
        ea16 = jnp.concatenate(
            [aux, jnp.ones((E, 1), jnp.float32),
             jnp.zeros((E, 11), jnp.float32)], axis=1)
        aux_p = _pad_rows(ea16, E_pad, 0)
    else:
        aux_p = _pad_rows(aux.reshape(-1), E_pad, 0)
    k = _edge_kernel(E_pad, y.shape[0], d, n_out_pad, not dst_split, mode, nb)
    res = k(y, s128, d128, aux_p)
    if mode == "conv2":
        res, res2 = res
        return (res[:n_dst], res[n_out_pad:n_out_pad + n_dst],
                res2[:n_dst], res2[n_out_pad:n_out_pad + n_dst])
    if dst_split:
        half = n_dst // 2
        return jnp.concatenate([res[:half], res[n_out_pad:n_out_pad + half]], 0)
    return res[:n_dst], res[n_out_pad:n_out_pad + n_dst]


def _z_edges(ea, We, b, d, E_pad):
    E = ea.shape[0]
    R = 4096
    Wp = _pad_cols(We, d)
    bp = _pad_cols(b.reshape(1, -1), d)
    grid = (pl.cdiv(E_pad, R),)
    last = (E - 1) // R

    def body(e_ref, w_ref, b_ref, o_ref):
        o_ref[...] = (jnp.dot(e_ref[...], w_ref[...],
                              preferred_element_type=jnp.float32) + b_ref[...])

    return pl.pallas_call(
        body, grid=grid,
        in_specs=[pl.BlockSpec((R, ea.shape[1]),
                               lambda i: (jnp.minimum(i, last), 0)),
                  pl.BlockSpec(Wp.shape, lambda i: (0, 0)),
                  pl.BlockSpec(bp.shape, lambda i: (0, 0))],
        out_specs=pl.BlockSpec((R, d), lambda i: (i, 0)),
        out_shape=jax.ShapeDtypeStruct((E_pad, d), jnp.float32),
    )(ea, Wp, bp)


def _wz(We, b, d):
    return jnp.concatenate([_pad_cols(We, d), _pad_cols(b.reshape(1, -1), d),
                            jnp.zeros((11, d), jnp.float32)], axis=0)


def kernel(xc, xf, ei_cf, ea_cf, ei_fp, ea_fp, ei_pp0, ea_pp0, ei_pp1, ea_pp1,
           ei_pp2, ea_pp2, ei_pp3, ea_pp3, ei_pc, ea_pc,
           ei_pool0, ea_pool0, ei_unpool0, ea_unpool0,
           ei_pool1, ea_pool1, ei_unpool1, ea_unpool1,
           ei_pool2, ea_pool2, ei_unpool2, ea_unpool2,
           W_cf, W_fp, Wm2, Ws2, Wm3, Ws3, Wm4a, Ws4a, Wm4b, Ws4b,
           Wm4c, Ws4c, Wm4d, Ws4d, Wm7, Ws7, Wm8, Ws8, Wm9, Ws9, W9b, Wf,
           b_cf, b_fp, bm2, bs2, bm3, bs3, bm4a, bs4a, bm4b, bs4b,
           bm4c, bs4c, bm4d, bs4d, bm7, bs7, bm8, bs8, bm9, bs9, b9b, bf):
    N_C, N_F = xc.shape[0], xf.shape[0]
    N_P0, N_P1, N_P2, N_P3 = 100000, 50000, 25000, 12500
    f32 = jnp.float32
    dot = lambda a, b: jnp.dot(a, b, preferred_element_type=f32)

    y_c = _tc(lambda x, w: dot(x, w), N_C, [16], [xc], [_pad_cols(W_cf[:2], 16)])
    z_cf = lambda ep: _z_edges(ea_cf, W_cf[2:6], b_cf, 16, ep)
    aggF = _edge_pass(y_c, ei_cf, z_cf, N_F, "conv", dst_split=True)

    Wh = _pad_rows(W_fp[:12], 16)
    Wx = W_fp[12:16]
    y_f = _tc(lambda a, x, wh, wx: dot(jnp.maximum(a, 0.), wh) + dot(x, wx),
              N_F, [16], [aggF, xf], [Wh, Wx])
    z_fp = lambda ep: _z_edges(ea_fp, W_fp[16:20], b_fp, 16, ep)
    q0, q1 = _edge_pass(y_f, ei_fp, z_fp, N_P0, "conv")
    c1 = _tc(lambda a, b: jnp.maximum(a + b, 0.), N_P0, [16], [q0, q1], [])

    q0, q1 = _edge_pass(c1, ei_pool0, ea_pool0, N_P1, "pool")
    y2, s2 = _tc(lambda a, b, wm, ws, bs: ((lambda p: (dot(p, wm), dot(p, ws) + bs))(a + b)),
                 N_P1, [16, 16], [q0, q1], [Wm2[:16], Ws2, bs2.reshape(1, -1)])
    z2 = lambda ep: _z_edges(ea_pp1, Wm2[16:20], bm2, 16, ep)
    q0, q1 = _edge_pass(y2, ei_pp1, z2, N_P1, "conv")
    c2 = _tc(lambda a, b, s: jnp.maximum(a + b + s, 0.), N_P1, [16], [q0, q1, s2], [])

    q0, q1 = _edge_pass(c2, ei_pool1, ea_pool1, N_P2, "pool")
    y3, s3 = _tc(lambda a, b, wm, ws, bs: ((lambda p: (dot(p, wm), dot(p, ws) + bs))(a + b)),
                 N_P2, [16, 16], [q0, q1], [Wm3[:16], Ws3, bs3.reshape(1, -1)])
    q0, q1, A0, A1 = _edge_pass(y3, ei_pp2, ea_pp2, N_P2, "conv2")
    c3 = _tc(lambda a, b, A, B, s, wz: jnp.maximum(a + b + dot(A + B, wz) + s, 0.),
             N_P2, [16], [q0, q1, A0, A1, s3], [_wz(Wm3[16:20], bm3, 16)])

    q0, q1 = _edge_pass(c3, ei_pool2, ea_pool2, N_P3, "pool")
    p = _tc(lambda a, b: a + b, N_P3, [16], [q0, q1], [])
    for Wm, Ws, bm, bs in ((Wm4a, Ws4a, bm4a, bs4a), (Wm4b, Ws4b, bm4b, bs4b),
                           (Wm4c, Ws4c, bm4c, bs4c), (Wm4d, Ws4d, bm4d, bs4d)):
        dx = Wm.shape[0] - 4
        y4, s4 = _tc(lambda p_, wm, ws, bs_: (dot(p_, wm), dot(p_, ws) + bs_),
                     N_P3, [32, 32], [p], [Wm[:dx], Ws, bs.reshape(1, -1)])
        q0, q1, A0, A1 = _edge_pass(y4, ei_pp3, ea_pp3, N_P3, "conv2")
        p = _tc(lambda a, b, A, B, s, wz: jnp.maximum(a + b + dot(A + B, wz) + s, 0.),
                N_P3, [32], [q0, q1, A0, A1, s4], [_wz(Wm[dx:], bm, 32)])

    q0, q1 = _edge_pass(p, ei_unpool2, ea_unpool2, N_P2, "pool")
    y7, s7 = _tc(lambda a, b, c_, wm1, wm2_, ws1, ws2_, bs_:
                 ((lambda u: (dot(u, wm1) + dot(c_, wm2_),
                              dot(u, ws1) + dot(c_, ws2_) + bs_))(a + b)),
                 N_P2, [32, 32], [q0, q1, c3],
                 [Wm7[:32], Wm7[32:48], Ws7[:32], Ws7[32:48], bs7.reshape(1, -1)])
    q0, q1, A0, A1 = _edge_pass(y7, ei_pp2, ea_pp2, N_P2, "conv2")
    h7 = _tc(lambda a, b, A, B, s, wz: jnp.maximum(a + b + dot(A + B, wz) + s, 0.),
             N_P2, [32], [q0, q1, A0, A1, s7], [_wz(Wm7[48:52], bm7, 32)])

    q0, q1 = _edge_pass(h7, ei_unpool1, ea_unpool1, N_P1, "pool")
    y8, s8 = _tc(lambda a, b, c_, wm1, wm2_, ws1, ws2_, bs_:
                 ((lambda u: (dot(u, wm1) + dot(c_, wm2_),
                              dot(u, ws1) + dot(c_, ws2_) + bs_))(a + b)),
                 N_P1, [16, 16], [q0, q1, c2],
                 [Wm8[:32], Wm8[32:48], Ws8[:32], Ws8[32:48], bs8.reshape(1, -1)])
    z8 = lambda ep: _z_edges(ea_pp1, Wm8[48:52], bm8, 16, ep)
    q0, q1 = _edge_pass(y8, ei_pp1, z8, N_P1, "conv")
    h8 = _tc(lambda a, b, s: jnp.maximum(a + b + s, 0.), N_P1, [16], [q0, q1, s8], [])

    q0, q1 = _edge_pass(h8, ei_unpool0, ea_unpool0, N_P0, "pool")
    y9, s9 = _tc(lambda a, b, c_, wm1, wm2_, ws1, ws2_, bs_:
                 ((lambda u: (dot(u, wm1) + dot(c_, wm2_),
                              dot(u, ws1) + dot(c_, ws2_) + bs_))(a + b)),
                 N_P0, [16, 16], [q0, q1, c1],
                 [Wm9[:16], Wm9[16:32], Ws9[:16], Ws9[16:32], bs9.reshape(1, -1)])
    z9 = lambda ep: _z_edges(ea_pp0, Wm9[32:36], bm9, 16, ep)
    q0, q1 = _edge_pass(y9, ei_pp0, z9, N_P0, "conv")

    y9b = _tc(lambda a, b, s, w: dot(jnp.maximum(a + b + s, 0.), w),
              N_P0, [16], [q0, q1, s9], [W9b[:16]])
    z9b = lambda ep: _z_edges(ea_pc, W9b[16:20], b9b, 16, ep)
    q0, q1 = _edge_pass(y9b, ei_pc, z9b, N_C, "conv")
    out = _tc(lambda a, b, wf, bf_: dot(jnp.maximum(a + b, 0.), wf) + bf_,
              N_C, [1], [q0, q1], [Wf, bf.reshape(1, -1)])
    return out


# speedup vs baseline: 1.2707x; 1.0085x over previous
import functools
import jax
import jax.numpy as jnp
from jax import lax
from jax.experimental import pallas as pl
from jax.experimental.pallas import tpu as pltpu
from jax.experimental.pallas import tpu_sc as plsc

NC, NS, L = 2, 16, 16
NW = NC * NS
CHB = 128
ZN = 256
SPMEM_WORDS = 2097151


def _pick_nb(d, n_out_pad, mode):
    acc_words = (n_out_pad + 8) * (d + (16 if mode == "conv2" else 0))
    budget = (SPMEM_WORDS - acc_words - 65536) // NS
    for nb in ((4, 2, 1) if mode == "pool" else (8, 4, 2, 1)):
        ch = nb * CHB
        aux_w = {"conv": ch * d, "conv2": ch * 16, "pool": ch}[mode]
        words = 2 * (ch * d + aux_w + 2 * ch) + ZN * d + (ZN * 16 if mode == "conv2" else 0)
        if words <= budget:
            return nb
    raise ValueError("accumulator too large for Spmem")


def _tc(fn, n_rows, out_dims, big, small, R=2048):
    grid = (pl.cdiv(n_rows, R),)
    nb_, ns_ = len(big), len(small)

    def body(*refs):
        vals = [r[...] for r in refs[:nb_ + ns_]]
        outs = fn(*vals)
        if not isinstance(outs, (tuple, list)):
            outs = (outs,)
        for r, o in zip(refs[nb_ + ns_:], outs):
            r[...] = o

    in_specs = ([pl.BlockSpec((R, a.shape[1]), lambda i: (i, 0)) for a in big]
                + [pl.BlockSpec(w.shape, lambda i: (0,) * w.ndim) for w in small])
    out_specs = [pl.BlockSpec((R, d), lambda i: (i, 0)) for d in out_dims]
    out_shape = [jax.ShapeDtypeStruct((n_rows, d), jnp.float32) for d in out_dims]
    res = pl.pallas_call(body, grid=grid, in_specs=in_specs,
                         out_specs=out_specs, out_shape=out_shape)(*big, *small)
    return res if len(out_dims) > 1 else res[0]


@functools.lru_cache(maxsize=None)
def _edge_kernel(E_pad, n_src, d, n_out_pad, edge_split, mode, NB):
    CH = NB * CHB
    EPT = E_pad // (NW if edge_split else NS)
    nch = EPT // CH
    rpt = n_out_pad // NS
    n_acc = n_out_pad + 8
    E128 = E_pad // CHB
    mesh = plsc.VectorSubcoreMesh(core_axis_name="c", subcore_axis_name="s",
                                  num_cores=NC, num_subcores=NS)
    aux_w = {"conv": d, "conv2": 16}.get(mode)
    aux_scr = (pltpu.VMEM((2, CH, aux_w), jnp.float32) if aux_w
               else pltpu.VMEM((2, CH), jnp.float32))
    conv2 = mode == "conv2"

    def body(*refs):
        if conv2:
            (y_hbm, s128, d128, aux_hbm, out_hbm, out2_hbm, acc, acc2,
             sidx, didx, rows, aux_v, zbuf, zbuf2, sem_i, sem_g, sem_s) = refs
        else:
            (y_hbm, s128, d128, aux_hbm, out_hbm, acc,
             sidx, didx, rows, aux_v, zbuf, sem_i, sem_g, sem_s) = refs
            out2_hbm = acc2 = zbuf2 = None
        c = lax.axis_index("c")
        s = lax.axis_index("s")
        zvec = jnp.zeros((L,), jnp.float32)

        def zb(r, _):
            for j in range(d // L):
                zbuf[r, pl.ds(j * L, L)] = zvec
            if conv2:
                zbuf2[r, :] = zvec
            return 0
        lax.fori_loop(0, ZN, zb, 0)

        base_r = s * rpt
        nzf, nzr = rpt // ZN, rpt % ZN

        def zero_drain(src_of, dst_of):
            ops = []
            for r in range(nzf):
                ops += list(zip(src_of(base_r + r * ZN, ZN),
                                dst_of(base_r + r * ZN, ZN)))
            if nzr:
                ops += list(zip(src_of(base_r + nzf * ZN, nzr),
                                dst_of(base_r + nzf * ZN, nzr)))
            for i in range(0, len(ops), 8):
                grp = [pltpu.async_copy(sr, dr, sem_g)
                       for sr, dr in ops[i:i + 8]]
                for g in grp:
                    g.wait()

        accs = [acc, acc2] if conv2 else [acc]
        zbufs = [zbuf, zbuf2] if conv2 else [zbuf]
        zero_drain(lambda o, n: [z.at[pl.ds(0, n)] for z in zbufs],
                   lambda o, n: [a.at[pl.ds(o, n)] for a in accs])

        @pl.when(s == 0)
        def _():
            for z, a in zip(zbufs, accs):
                pltpu.sync_copy(z.at[pl.ds(0, 8)], a.at[pl.ds(n_out_pad, 8)])

        plsc.subcore_barrier()

        if edge_split:
            base128 = (c * NS + s) * (EPT // CHB)
        else:
            base128 = s * (EPT // CHB)

        def scatter_ops(b):
            ops = [(rows.at[b, pl.ds(j * CHB, CHB)], acc.at[didx.at[b, j]])
                   for j in range(NB)]
            if mode == "conv":
                ops += [(aux_v.at[b, pl.ds(j * CHB, CHB)], acc.at[didx.at[b, j]])
                        for j in range(NB)]
            elif conv2:
                ops += [(aux_v.at[b, pl.ds(j * CHB, CHB)], acc2.at[didx.at[b, j]])
                        for j in range(NB)]
            return ops

        def phase(i, b):
            @pl.when(i >= 1)
            def _():
                pltpu.make_async_copy(s128.at[pl.ds(0, NB)], sidx.at[b], sem_i).wait()
                pltpu.make_async_copy(s128.at[pl.ds(0, NB)], didx.at[b], sem_i).wait()
            off128 = base128 + i * NB
            gs = [pltpu.async_copy(y_hbm.at[sidx.at[b, j]],
                                   rows.at[b, pl.ds(j * CHB, CHB)], sem_g)
                  for j in range(NB)]
            gs.append(pltpu.async_copy(aux_hbm.at[pl.ds(off128 * CHB, CH)],
                                       aux_v.at[b], sem_g))
            @pl.when(i >= 1)
            def _():
                for src, dst in scatter_ops(1 - b):
                    pltpu.make_async_copy(src, dst, sem_s).wait()
            @pl.when(i + 1 < nch)
            def _():
                offn = base128 + (i + 1) * NB
                pltpu.async_copy(s128.at[pl.ds(offn, NB)], sidx.at[1 - b], sem_i)
                pltpu.async_copy(d128.at[pl.ds(c * E128 + offn, NB)],
                                 didx.at[1 - b], sem_i)
            for g in gs:
                g.wait()
            if mode == "pool":
                @plsc.parallel_loop(0, CH, 1, unroll=4)
                def pm(e):
                    b16 = plsc.load_gather(aux_v.at[b],
                                           [jnp.full((L,), e, jnp.int32)])
                    for j in range(d // L):
                        rows[b, e, pl.ds(j * L, L)] = (
                            rows[b, e, pl.ds(j * L, L)] * b16)
            for src, dst in scatter_ops(b):
                pltpu.async_copy(src, dst, sem_s, add=True)

        def chunk2(i2, _):
            phase(2 * i2, 0)
            phase(2 * i2 + 1, 1)
            return 0
        pltpu.sync_copy(s128.at[pl.ds(base128, NB)], sidx.at[0])
        pltpu.sync_copy(d128.at[pl.ds(c * E128 + base128, NB)], didx.at[0])
        lax.fori_loop(0, nch // 2, chunk2, 0)
        for src, dst in scatter_ops((nch - 1) % 2):
            pltpu.make_async_copy(src, dst, sem_s).wait()

        plsc.subcore_barrier()

        outs = [out_hbm, out2_hbm] if conv2 else [out_hbm]
        zero_drain(lambda o, n: [a.at[pl.ds(o, n)] for a in accs],
                   lambda o, n: [h.at[pl.ds(c * n_out_pad + o, n)] for h in outs])

    out_type = jax.ShapeDtypeStruct((NC * n_out_pad, d), jnp.float32)
    if conv2:
        out_type = (out_type,
                    jax.ShapeDtypeStruct((NC * n_out_pad, 16), jnp.float32))
    scratch = [pltpu.VMEM_SHARED((n_acc, d), jnp.float32)]
    if conv2:
        scratch.append(pltpu.VMEM_SHARED((n_acc, 16), jnp.float32))
    scratch += [
        pltpu.VMEM((2, NB, CHB), jnp.int32),
        pltpu.VMEM((2, NB, CHB), jnp.int32),
        pltpu.VMEM((2, CH, d), jnp.float32),
        aux_scr,
        pltpu.VMEM((ZN, d), jnp.float32),
    ]
    if conv2:
        scratch.append(pltpu.VMEM((ZN, 16), jnp.float32))
    scratch += [pltpu.SemaphoreType.DMA] * 3
    return pl.kernel(
        body,
        out_type=out_type,
        mesh=mesh,
        compiler_params=pltpu.CompilerParams(use_tc_tiling_on_sc=False,
                                             needs_layout_passes=False),
        scratch_types=scratch,
        name=f"edge_{mode}_{E_pad}_{n_src}_{d}_{n_out_pad}_{int(edge_split)}",
    )


def _ceil_to(x, m):
    return (x + m - 1) // m * m


def _pad_rows(a, n, val=0):
    if a.shape[0] == n:
        return a
    pad = [(0, n - a.shape[0])] + [(0, 0)] * (a.ndim - 1)
    return jnp.pad(a, pad, constant_values=val)


def _pad_cols(w, d):
    if w.shape[1] == d:
        return w
    return jnp.pad(w, [(0, 0), (0, d - w.shape[1])])


def _prep_edges(ei, n_dst_pad, E_pad, dst_split_half=None):
    src = _pad_rows(ei[0], E_pad, 0)
    if dst_split_half is None:
        dst = _pad_rows(ei[1], E_pad, n_dst_pad)
        d2 = jnp.stack([dst, dst])
    else:
        h = dst_split_half
        dst = _pad_rows(ei[1], E_pad, 2 * h)
        halves = []
        for c in range(2):
            dl = dst - c * h
            halves.append(jnp.where((dl >= 0) & (dl < h), dl, h))
        d2 = jnp.stack(halves)
    return src.reshape(E_pad // CHB, CHB), d2.reshape(2 * E_pad // CHB, CHB)


def _edge_pass(y, ei, aux, n_dst, mode, dst_split=False):
    E = ei.shape[1]
    d = y.shape[1]
    if dst_split:
        half = n_dst // 2
        n_out_pad = _ceil_to(half, NS * 8)
        nb = _pick_nb(d, n_out_pad, mode)
        E_pad = _ceil_to(E, NW * nb * CHB * 2)
        s128, d128 = _prep_edges(ei, None, E_pad, dst_split_half=half)
    else:
        n_out_pad = _ceil_to(n_dst, NS * 8)
        nb = _pick_nb(d, n_out_pad, mode)
        E_pad = _ceil_to(E, NW * nb * CHB * 2)
        s128, d128 = _prep_edges(ei, n_out_pad, E_pad)
    if mode == "conv":
        aux_p = aux(E_pad)
    elif mode == "conv2":
        ea16 = jnp.concatenate(
            [aux, jnp.ones((E, 1), jnp.float32),
             jnp.zeros((E, 11), jnp.float32)], axis=1)
        aux_p = _pad_rows(ea16, E_pad, 0)
    else:
        aux_p = _pad_rows(aux.reshape(-1), E_pad, 0)
    k = _edge_kernel(E_pad, y.shape[0], d, n_out_pad, not dst_split, mode, nb)
    res = k(y, s128, d128, aux_p)
    if mode == "conv2":
        res, res2 = res
        return (res[:n_dst], res[n_out_pad:n_out_pad + n_dst],
                res2[:n_dst], res2[n_out_pad:n_out_pad + n_dst])
    if dst_split:
        half = n_dst // 2
        return jnp.concatenate([res[:half], res[n_out_pad:n_out_pad + half]], 0)
    return res[:n_dst], res[n_out_pad:n_out_pad + n_dst]


def _z_edges(ea, We, b, d, E_pad):
    E = ea.shape[0]
    R = 4096
    Wp = _pad_cols(We, d)
    bp = _pad_cols(b.reshape(1, -1), d)
    grid = (pl.cdiv(E_pad, R),)
    last = (E - 1) // R

    def body(e_ref, w_ref, b_ref, o_ref):
        o_ref[...] = (jnp.dot(e_ref[...], w_ref[...],
                              preferred_element_type=jnp.float32) + b_ref[...])

    return pl.pallas_call(
        body, grid=grid,
        in_specs=[pl.BlockSpec((R, ea.shape[1]),
                               lambda i: (jnp.minimum(i, last), 0)),
                  pl.BlockSpec(Wp.shape, lambda i: (0, 0)),
                  pl.BlockSpec(bp.shape, lambda i: (0, 0))],
        out_specs=pl.BlockSpec((R, d), lambda i: (i, 0)),
        out_shape=jax.ShapeDtypeStruct((E_pad, d), jnp.float32),
    )(ea, Wp, bp)


def _wz(We, b, d):
    return jnp.concatenate([_pad_cols(We, d), _pad_cols(b.reshape(1, -1), d),
                            jnp.zeros((11, d), jnp.float32)], axis=0)


def kernel(xc, xf, ei_cf, ea_cf, ei_fp, ea_fp, ei_pp0, ea_pp0, ei_pp1, ea_pp1,
           ei_pp2, ea_pp2, ei_pp3, ea_pp3, ei_pc, ea_pc,
           ei_pool0, ea_pool0, ei_unpool0, ea_unpool0,
           ei_pool1, ea_pool1, ei_unpool1, ea_unpool1,
           ei_pool2, ea_pool2, ei_unpool2, ea_unpool2,
           W_cf, W_fp, Wm2, Ws2, Wm3, Ws3, Wm4a, Ws4a, Wm4b, Ws4b,
           Wm4c, Ws4c, Wm4d, Ws4d, Wm7, Ws7, Wm8, Ws8, Wm9, Ws9, W9b, Wf,
           b_cf, b_fp, bm2, bs2, bm3, bs3, bm4a, bs4a, bm4b, bs4b,
           bm4c, bs4c, bm4d, bs4d, bm7, bs7, bm8, bs8, bm9, bs9, b9b, bf):
    N_C, N_F = xc.shape[0], xf.shape[0]
    N_P0, N_P1, N_P2, N_P3 = 100000, 50000, 25000, 12500
    f32 = jnp.float32
    dot = lambda a, b: jnp.dot(a, b, preferred_element_type=f32)

    y_c = _tc(lambda x, w: dot(x, w), N_C, [16], [xc], [_pad_cols(W_cf[:2], 16)])
    z_cf = lambda ep: _z_edges(ea_cf, W_cf[2:6], b_cf, 16, ep)
    aggF = _edge_pass(y_c, ei_cf, z_cf, N_F, "conv", dst_split=True)

    Wh = _pad_rows(W_fp[:12], 16)
    Wx = W_fp[12:16]
    y_f = _tc(lambda a, x, wh, wx: dot(jnp.maximum(a, 0.), wh) + dot(x, wx),
              N_F, [16], [aggF, xf], [Wh, Wx])
    z_fp = lambda ep: _z_edges(ea_fp, W_fp[16:20], b_fp, 16, ep)
    q0, q1 = _edge_pass(y_f, ei_fp, z_fp, N_P0, "conv")
    c1 = _tc(lambda a, b: jnp.maximum(a + b, 0.), N_P0, [16], [q0, q1], [])

    q0, q1 = _edge_pass(c1, ei_pool0, ea_pool0, N_P1, "pool")
    y2, s2 = _tc(lambda a, b, wm, ws, bs: ((lambda p: (dot(p, wm), dot(p, ws) + bs))(a + b)),
                 N_P1, [16, 16], [q0, q1], [Wm2[:16], Ws2, bs2.reshape(1, -1)])
    z2 = lambda ep: _z_edges(ea_pp1, Wm2[16:20], bm2, 16, ep)
    q0, q1 = _edge_pass(y2, ei_pp1, z2, N_P1, "conv")
    c2 = _tc(lambda a, b, s: jnp.maximum(a + b + s, 0.), N_P1, [16], [q0, q1, s2], [])

    q0, q1 = _edge_pass(c2, ei_pool1, ea_pool1, N_P2, "pool")
    y3, s3 = _tc(lambda a, b, wm, ws, bs: ((lambda p: (dot(p, wm), dot(p, ws) + bs))(a + b)),
                 N_P2, [16, 16], [q0, q1], [Wm3[:16], Ws3, bs3.reshape(1, -1)])
    q0, q1, A0, A1 = _edge_pass(y3, ei_pp2, ea_pp2, N_P2, "conv2")
    c3 = _tc(lambda a, b, A, B, s, wz: jnp.maximum(a + b + dot(A + B, wz) + s, 0.),
             N_P2, [16], [q0, q1, A0, A1, s3], [_wz(Wm3[16:20], bm3, 16)])

    q0, q1 = _edge_pass(c3, ei_pool2, ea_pool2, N_P3, "pool")
    p = _tc(lambda a, b: a + b, N_P3, [16], [q0, q1], [])
    for Wm, Ws, bm, bs in ((Wm4a, Ws4a, bm4a, bs4a), (Wm4b, Ws4b, bm4b, bs4b),
                           (Wm4c, Ws4c, bm4c, bs4c), (Wm4d, Ws4d, bm4d, bs4d)):
        dx = Wm.shape[0] - 4
        y4, s4 = _tc(lambda p_, wm, ws, bs_: (dot(p_, wm), dot(p_, ws) + bs_),
                     N_P3, [32, 32], [p], [Wm[:dx], Ws, bs.reshape(1, -1)])
        q0, q1, A0, A1 = _edge_pass(y4, ei_pp3, ea_pp3, N_P3, "conv2")
        p = _tc(lambda a, b, A, B, s, wz: jnp.maximum(a + b + dot(A + B, wz) + s, 0.),
                N_P3, [32], [q0, q1, A0, A1, s4], [_wz(Wm[dx:], bm, 32)])

    q0, q1 = _edge_pass(p, ei_unpool2, ea_unpool2, N_P2, "pool")
    y7, s7 = _tc(lambda a, b, c_, wm1, wm2_, ws1, ws2_, bs_:
                 ((lambda u: (dot(u, wm1) + dot(c_, wm2_),
                              dot(u, ws1) + dot(c_, ws2_) + bs_))(a + b)),
                 N_P2, [32, 32], [q0, q1, c3],
                 [Wm7[:32], Wm7[32:48], Ws7[:32], Ws7[32:48], bs7.reshape(1, -1)])
    q0, q1, A0, A1 = _edge_pass(y7, ei_pp2, ea_pp2, N_P2, "conv2")
    h7 = _tc(lambda a, b, A, B, s, wz: jnp.maximum(a + b + dot(A + B, wz) + s, 0.),
             N_P2, [32], [q0, q1, A0, A1, s7], [_wz(Wm7[48:52], bm7, 32)])

    q0, q1 = _edge_pass(h7, ei_unpool1, ea_unpool1, N_P1, "pool")
    y8, s8 = _tc(lambda a, b, c_, wm1, wm2_, ws1, ws2_, bs_:
                 ((lambda u: (dot(u, wm1) + dot(c_, wm2_),
                              dot(u, ws1) + dot(c_, ws2_) + bs_))(a + b)),
                 N_P1, [16, 16], [q0, q1, c2],
                 [Wm8[:32], Wm8[32:48], Ws8[:32], Ws8[32:48], bs8.reshape(1, -1)])
    z8 = lambda ep: _z_edges(ea_pp1, Wm8[48:52], bm8, 16, ep)
    q0, q1 = _edge_pass(y8, ei_pp1, z8, N_P1, "conv")
    h8 = _tc(lambda a, b, s: jnp.maximum(a + b + s, 0.), N_P1, [16], [q0, q1, s8], [])

    q0, q1 = _edge_pass(h8, ei_unpool0, ea_unpool0, N_P0, "pool")
    y9, s9 = _tc(lambda a, b, c_, wm1, wm2_, ws1, ws2_, bs_:
                 ((lambda u: (dot(u, wm1) + dot(c_, wm2_),
                              dot(u, ws1) + dot(c_, ws2_) + bs_))(a + b)),
                 N_P0, [16, 16], [q0, q1, c1],
                 [Wm9[:16], Wm9[16:32], Ws9[:16], Ws9[16:32], bs9.reshape(1, -1)])
    z9 = lambda ep: _z_edges(ea_pp0, Wm9[32:36], bm9, 16, ep)
    q0, q1 = _edge_pass(y9, ei_pp0, z9, N_P0, "conv")

    y9b = _tc(lambda a, b, s, w: dot(jnp.maximum(a + b + s, 0.), w),
              N_P0, [16], [q0, q1, s9], [W9b[:16]])
    z9b = lambda ep: _z_edges(ea_pc, W9b[16:20], b9b, 16, ep)
    q0, q1 = _edge_pass(y9b, ei_pc, z9b, N_C, "conv")
    out = _tc(lambda a, b, wf, bf_: dot(jnp.maximum(a + b, 0.), wf) + bf_,
              N_C, [1], [q0, q1], [Wf, bf.reshape(1, -1)])
    return out


# speedup vs baseline: 1.3324x; 1.0485x over previous
import functools
import jax
import jax.numpy as jnp
from jax import lax
from jax.experimental import pallas as pl
from jax.experimental.pallas import tpu as pltpu
from jax.experimental.pallas import tpu_sc as plsc

NC, NS, L = 2, 16, 16
NW = NC * NS
CHB = 128
ZN = 256
SPMEM_WORDS = 2097151


def _pick_nb(d, n_out_pad, mode):
    acc_words = (n_out_pad + 8) * (d + (16 if mode == "conv2" else 0))
    budget = (SPMEM_WORDS - acc_words - 65536) // NS
    for nb in ((4, 2, 1) if mode == "pool" else (8, 4, 2, 1)):
        ch = nb * CHB
        aux_w = {"conv": ch * d, "conv2": ch * 16, "pool": ch}[mode]
        words = 2 * (ch * d + aux_w + 2 * ch) + ZN * d + (ZN * 16 if mode == "conv2" else 0)
        if words <= budget:
            return nb
    raise ValueError("accumulator too large for Spmem")


def _tc(fn, n_rows, out_dims, big, small, R=2048):
    grid = (pl.cdiv(n_rows, R),)
    nb_, ns_ = len(big), len(small)

    def body(*refs):
        vals = [r[...] for r in refs[:nb_ + ns_]]
        outs = fn(*vals)
        if not isinstance(outs, (tuple, list)):
            outs = (outs,)
        for r, o in zip(refs[nb_ + ns_:], outs):
            r[...] = o

    in_specs = ([pl.BlockSpec((R, a.shape[1]), lambda i: (i, 0)) for a in big]
                + [pl.BlockSpec(w.shape, lambda i: (0,) * w.ndim) for w in small])
    out_specs = [pl.BlockSpec((R, d), lambda i: (i, 0)) for d in out_dims]
    out_shape = [jax.ShapeDtypeStruct((n_rows, d), jnp.float32) for d in out_dims]
    res = pl.pallas_call(
        body, grid=grid, in_specs=in_specs,
        out_specs=out_specs, out_shape=out_shape,
        compiler_params=pltpu.CompilerParams(
            allow_input_fusion=[True] * (nb_ + ns_)),
    )(*big, *small)
    return res if len(out_dims) > 1 else res[0]


@functools.lru_cache(maxsize=None)
def _edge_kernel(E_pad, n_src, d, n_out_pad, edge_split, mode, NB):
    CH = NB * CHB
    EPT = E_pad // (NW if edge_split else NS)
    nch = EPT // CH
    rpt = n_out_pad // NS
    n_acc = n_out_pad + 8
    E128 = E_pad // CHB
    mesh = plsc.VectorSubcoreMesh(core_axis_name="c", subcore_axis_name="s",
                                  num_cores=NC, num_subcores=NS)
    aux_w = {"conv": d, "conv2": 16}.get(mode)
    aux_scr = (pltpu.VMEM((2, CH, aux_w), jnp.float32) if aux_w
               else pltpu.VMEM((2, CH), jnp.float32))
    conv2 = mode == "conv2"

    def body(*refs):
        if conv2:
            (y_hbm, s128, d128, aux_hbm, out_hbm, out2_hbm, acc, acc2,
             sidx, didx, rows, aux_v, zbuf, zbuf2, sem_i, sem_g, sem_s) = refs
        else:
            (y_hbm, s128, d128, aux_hbm, out_hbm, acc,
             sidx, didx, rows, aux_v, zbuf, sem_i, sem_g, sem_s) = refs
            out2_hbm = acc2 = zbuf2 = None
        c = lax.axis_index("c")
        s = lax.axis_index("s")
        zvec = jnp.zeros((L,), jnp.float32)

        def zb(r, _):
            for j in range(d // L):
                zbuf[r, pl.ds(j * L, L)] = zvec
            if conv2:
                zbuf2[r, :] = zvec
            return 0
        lax.fori_loop(0, ZN, zb, 0)

        base_r = s * rpt
        nzf, nzr = rpt // ZN, rpt % ZN

        def zero_drain(src_of, dst_of):
            ops = []
            for r in range(nzf):
                ops += list(zip(src_of(base_r + r * ZN, ZN),
                                dst_of(base_r + r * ZN, ZN)))
            if nzr:
                ops += list(zip(src_of(base_r + nzf * ZN, nzr),
                                dst_of(base_r + nzf * ZN, nzr)))
            for i in range(0, len(ops), 8):
                grp = [pltpu.async_copy(sr, dr, sem_g)
                       for sr, dr in ops[i:i + 8]]
                for g in grp:
                    g.wait()

        accs = [acc, acc2] if conv2 else [acc]
        zbufs = [zbuf, zbuf2] if conv2 else [zbuf]
        zero_drain(lambda o, n: [z.at[pl.ds(0, n)] for z in zbufs],
                   lambda o, n: [a.at[pl.ds(o, n)] for a in accs])

        @pl.when(s == 0)
        def _():
            for z, a in zip(zbufs, accs):
                pltpu.sync_copy(z.at[pl.ds(0, 8)], a.at[pl.ds(n_out_pad, 8)])

        plsc.subcore_barrier()

        if edge_split:
            base128 = (c * NS + s) * (EPT // CHB)
        else:
            base128 = s * (EPT // CHB)

        def scatter_ops(b):
            ops = [(rows.at[b, pl.ds(j * CHB, CHB)], acc.at[didx.at[b, j]])
                   for j in range(NB)]
            if mode == "conv":
                ops += [(aux_v.at[b, pl.ds(j * CHB, CHB)], acc.at[didx.at[b, j]])
                        for j in range(NB)]
            elif conv2:
                ops += [(aux_v.at[b, pl.ds(j * CHB, CHB)], acc2.at[didx.at[b, j]])
                        for j in range(NB)]
            return ops

        def phase(i, b):
            @pl.when(i >= 1)
            def _():
                pltpu.make_async_copy(s128.at[pl.ds(0, NB)], sidx.at[b], sem_i).wait()
                pltpu.make_async_copy(s128.at[pl.ds(0, NB)], didx.at[b], sem_i).wait()
            off128 = base128 + i * NB
            gs = [pltpu.async_copy(y_hbm.at[sidx.at[b, j]],
                                   rows.at[b, pl.ds(j * CHB, CHB)], sem_g)
                  for j in range(NB)]
            gs.append(pltpu.async_copy(aux_hbm.at[pl.ds(off128 * CHB, CH)],
                                       aux_v.at[b], sem_g))
            @pl.when(i >= 1)
            def _():
                for src, dst in scatter_ops(1 - b):
                    pltpu.make_async_copy(src, dst, sem_s).wait()
            @pl.when(i + 1 < nch)
            def _():
                offn = base128 + (i + 1) * NB
                pltpu.async_copy(s128.at[pl.ds(offn, NB)], sidx.at[1 - b], sem_i)
                pltpu.async_copy(d128.at[pl.ds(c * E128 + offn, NB)],
                                 didx.at[1 - b], sem_i)
            for g in gs:
                g.wait()
            if mode == "pool":
                @plsc.parallel_loop(0, CH, 1, unroll=4)
                def pm(e):
                    b16 = plsc.load_gather(aux_v.at[b],
                                           [jnp.full((L,), e, jnp.int32)])
                    for j in range(d // L):
                        rows[b, e, pl.ds(j * L, L)] = (
                            rows[b, e, pl.ds(j * L, L)] * b16)
            for src, dst in scatter_ops(b):
                pltpu.async_copy(src, dst, sem_s, add=True)

        def chunk2(i2, _):
            phase(2 * i2, 0)
            phase(2 * i2 + 1, 1)
            return 0
        pltpu.sync_copy(s128.at[pl.ds(base128, NB)], sidx.at[0])
        pltpu.sync_copy(d128.at[pl.ds(c * E128 + base128, NB)], didx.at[0])
        lax.fori_loop(0, nch // 2, chunk2, 0)
        for src, dst in scatter_ops((nch - 1) % 2):
            pltpu.make_async_copy(src, dst, sem_s).wait()

        plsc.subcore_barrier()

        outs = [out_hbm, out2_hbm] if conv2 else [out_hbm]
        zero_drain(lambda o, n: [a.at[pl.ds(o, n)] for a in accs],
                   lambda o, n: [h.at[pl.ds(c * n_out_pad + o, n)] for h in outs])

    out_type = jax.ShapeDtypeStruct((NC * n_out_pad, d), jnp.float32)
    if conv2:
        out_type = (out_type,
                    jax.ShapeDtypeStruct((NC * n_out_pad, 16), jnp.float32))
    scratch = [pltpu.VMEM_SHARED((n_acc, d), jnp.float32)]
    if conv2:
        scratch.append(pltpu.VMEM_SHARED((n_acc, 16), jnp.float32))
    scratch += [
        pltpu.VMEM((2, NB, CHB), jnp.int32),
        pltpu.VMEM((2, NB, CHB), jnp.int32),
        pltpu.VMEM((2, CH, d), jnp.float32),
        aux_scr,
        pltpu.VMEM((ZN, d), jnp.float32),
    ]
    if conv2:
        scratch.append(pltpu.VMEM((ZN, 16), jnp.float32))
    scratch += [pltpu.SemaphoreType.DMA] * 3
    return pl.kernel(
        body,
        out_type=out_type,
        mesh=mesh,
        compiler_params=pltpu.CompilerParams(use_tc_tiling_on_sc=False,
                                             needs_layout_passes=False),
        scratch_types=scratch,
        name=f"edge_{mode}_{E_pad}_{n_src}_{d}_{n_out_pad}_{int(edge_split)}",
    )


def _ceil_to(x, m):
    return (x + m - 1) // m * m


def _pad_rows(a, n, val=0):
    if a.shape[0] == n:
        return a
    pad = [(0, n - a.shape[0])] + [(0, 0)] * (a.ndim - 1)
    return jnp.pad(a, pad, constant_values=val)


def _pad_cols(w, d):
    if w.shape[1] == d:
        return w
    return jnp.pad(w, [(0, 0), (0, d - w.shape[1])])


def _prep_edges(ei, n_dst_pad, E_pad, dst_split_half=None):
    src = _pad_rows(ei[0], E_pad, 0)
    if dst_split_half is None:
        dst = _pad_rows(ei[1], E_pad, n_dst_pad)
        d2 = jnp.stack([dst, dst])
    else:
        h = dst_split_half
        dst = _pad_rows(ei[1], E_pad, 2 * h)
        halves = []
        for c in range(2):
            dl = dst - c * h
            halves.append(jnp.where((dl >= 0) & (dl < h), dl, h))
        d2 = jnp.stack(halves)
    return src.reshape(E_pad // CHB, CHB), d2.reshape(2 * E_pad // CHB, CHB)


def _edge_pass(y, ei, aux, n_dst, mode, dst_split=False):
    E = ei.shape[1]
    d = y.shape[1]
    if dst_split:
        half = n_dst // 2
        n_out_pad = _ceil_to(half, NS * 8)
        nb = _pick_nb(d, n_out_pad, mode)
        E_pad = _ceil_to(E, NW * nb * CHB * 2)
        s128, d128 = _prep_edges(ei, None, E_pad, dst_split_half=half)
    else:
        n_out_pad = _ceil_to(n_dst, NS * 8)
        nb = _pick_nb(d, n_out_pad, mode)
        E_pad = _ceil_to(E, NW * nb * CHB * 2)
        s128, d128 = _prep_edges(ei, n_out_pad, E_pad)
    if mode == "conv":
        aux_p = aux(E_pad)
    elif mode == "conv2":
        ea16 = jnp.concatenate(
            [aux, jnp.ones((E, 1), jnp.float32),
             jnp.zeros((E, 11), jnp.float32)], axis=1)
        aux_p = _pad_rows(ea16, E_pad, 0)
    else:
        aux_p = _pad_rows(aux.reshape(-1), E_pad, 0)
    k = _edge_kernel(E_pad, y.shape[0], d, n_out_pad, not dst_split, mode, nb)
    res = k(y, s128, d128, aux_p)
    if mode == "conv2":
        res, res2 = res
        return (res[:n_dst], res[n_out_pad:n_out_pad + n_dst],
                res2[:n_dst], res2[n_out_pad:n_out_pad + n_dst])
    if dst_split:
        half = n_dst // 2
        return jnp.concatenate([res[:half], res[n_out_pad:n_out_pad + half]], 0)
    return res[:n_dst], res[n_out_pad:n_out_pad + n_dst]


def _z_edges(ea, We, b, d, E_pad):
    E = ea.shape[0]
    R = 4096
    Wp = _pad_cols(We, d)
    bp = _pad_cols(b.reshape(1, -1), d)
    grid = (pl.cdiv(E_pad, R),)
    last = (E - 1) // R

    def body(e_ref, w_ref, b_ref, o_ref):
        o_ref[...] = (jnp.dot(e_ref[...], w_ref[...],
                              preferred_element_type=jnp.float32) + b_ref[...])

    return pl.pallas_call(
        body, grid=grid,
        in_specs=[pl.BlockSpec((R, ea.shape[1]),
                               lambda i: (jnp.minimum(i, last), 0)),
                  pl.BlockSpec(Wp.shape, lambda i: (0, 0)),
                  pl.BlockSpec(bp.shape, lambda i: (0, 0))],
        out_specs=pl.BlockSpec((R, d), lambda i: (i, 0)),
        out_shape=jax.ShapeDtypeStruct((E_pad, d), jnp.float32),
    )(ea, Wp, bp)


def _wz(We, b, d):
    return jnp.concatenate([_pad_cols(We, d), _pad_cols(b.reshape(1, -1), d),
                            jnp.zeros((11, d), jnp.float32)], axis=0)


def kernel(xc, xf, ei_cf, ea_cf, ei_fp, ea_fp, ei_pp0, ea_pp0, ei_pp1, ea_pp1,
           ei_pp2, ea_pp2, ei_pp3, ea_pp3, ei_pc, ea_pc,
           ei_pool0, ea_pool0, ei_unpool0, ea_unpool0,
           ei_pool1, ea_pool1, ei_unpool1, ea_unpool1,
           ei_pool2, ea_pool2, ei_unpool2, ea_unpool2,
           W_cf, W_fp, Wm2, Ws2, Wm3, Ws3, Wm4a, Ws4a, Wm4b, Ws4b,
           Wm4c, Ws4c, Wm4d, Ws4d, Wm7, Ws7, Wm8, Ws8, Wm9, Ws9, W9b, Wf,
           b_cf, b_fp, bm2, bs2, bm3, bs3, bm4a, bs4a, bm4b, bs4b,
           bm4c, bs4c, bm4d, bs4d, bm7, bs7, bm8, bs8, bm9, bs9, b9b, bf):
    N_C, N_F = xc.shape[0], xf.shape[0]
    N_P0, N_P1, N_P2, N_P3 = 100000, 50000, 25000, 12500
    f32 = jnp.float32
    dot = lambda a, b: jnp.dot(a, b, preferred_element_type=f32)

    y_c = _tc(lambda x, w: dot(x, w), N_C, [16], [xc], [_pad_cols(W_cf[:2], 16)])
    z_cf = lambda ep: _z_edges(ea_cf, W_cf[2:6], b_cf, 16, ep)
    aggF = _edge_pass(y_c, ei_cf, z_cf, N_F, "conv", dst_split=True)

    Wh = _pad_rows(W_fp[:12], 16)
    Wx = W_fp[12:16]
    y_f = _tc(lambda a, x, wh, wx: dot(jnp.maximum(a, 0.), wh) + dot(x, wx),
              N_F, [16], [aggF, xf], [Wh, Wx])
    z_fp = lambda ep: _z_edges(ea_fp, W_fp[16:20], b_fp, 16, ep)
    q0, q1 = _edge_pass(y_f, ei_fp, z_fp, N_P0, "conv")
    c1 = _tc(lambda a, b: jnp.maximum(a + b, 0.), N_P0, [16], [q0, q1], [])

    q0, q1 = _edge_pass(c1, ei_pool0, ea_pool0, N_P1, "pool")
    y2, s2 = _tc(lambda a, b, wm, ws, bs: ((lambda p: (dot(p, wm), dot(p, ws) + bs))(a + b)),
                 N_P1, [16, 16], [q0, q1], [Wm2[:16], Ws2, bs2.reshape(1, -1)])
    z2 = lambda ep: _z_edges(ea_pp1, Wm2[16:20], bm2, 16, ep)
    q0, q1 = _edge_pass(y2, ei_pp1, z2, N_P1, "conv")
    c2 = _tc(lambda a, b, s: jnp.maximum(a + b + s, 0.), N_P1, [16], [q0, q1, s2], [])

    q0, q1 = _edge_pass(c2, ei_pool1, ea_pool1, N_P2, "pool")
    y3, s3 = _tc(lambda a, b, wm, ws, bs: ((lambda p: (dot(p, wm), dot(p, ws) + bs))(a + b)),
                 N_P2, [16, 16], [q0, q1], [Wm3[:16], Ws3, bs3.reshape(1, -1)])
    q0, q1, A0, A1 = _edge_pass(y3, ei_pp2, ea_pp2, N_P2, "conv2")
    c3 = _tc(lambda a, b, A, B, s, wz: jnp.maximum(a + b + dot(A + B, wz) + s, 0.),
             N_P2, [16], [q0, q1, A0, A1, s3], [_wz(Wm3[16:20], bm3, 16)])

    q0, q1 = _edge_pass(c3, ei_pool2, ea_pool2, N_P3, "pool")
    p = _tc(lambda a, b: a + b, N_P3, [16], [q0, q1], [])
    pend = None
    for Wm, Ws, bm, bs in ((Wm4a, Ws4a, bm4a, bs4a), (Wm4b, Ws4b, bm4b, bs4b),
                           (Wm4c, Ws4c, bm4c, bs4c), (Wm4d, Ws4d, bm4d, bs4d)):
        dx = Wm.shape[0] - 4
        if pend is None:
            y4, s4 = _tc(lambda p_, wm, ws, bs_: (dot(p_, wm), dot(p_, ws) + bs_),
                         N_P3, [32, 32], [p], [Wm[:dx], Ws, bs.reshape(1, -1)])
        else:
            y4, s4 = _tc(
                lambda a, b, A, B, s, wz, wm, ws, bs_:
                ((lambda h: (dot(h, wm), dot(h, ws) + bs_))(
                    jnp.maximum(a + b + dot(A + B, wz) + s, 0.))),
                N_P3, [32, 32], pend[:5],
                [pend[5], Wm[:dx], Ws, bs.reshape(1, -1)])
        q0, q1, A0, A1 = _edge_pass(y4, ei_pp3, ea_pp3, N_P3, "conv2")
        pend = (q0, q1, A0, A1, s4, _wz(Wm[dx:], bm, 32))
    p = _tc(lambda a, b, A, B, s, wz: jnp.maximum(a + b + dot(A + B, wz) + s, 0.),
            N_P3, [32], pend[:5], [pend[5]])

    q0, q1 = _edge_pass(p, ei_unpool2, ea_unpool2, N_P2, "pool")
    y7, s7 = _tc(lambda a, b, c_, wm1, wm2_, ws1, ws2_, bs_:
                 ((lambda u: (dot(u, wm1) + dot(c_, wm2_),
                              dot(u, ws1) + dot(c_, ws2_) + bs_))(a + b)),
                 N_P2, [32, 32], [q0, q1, c3],
                 [Wm7[:32], Wm7[32:48], Ws7[:32], Ws7[32:48], bs7.reshape(1, -1)])
    q0, q1, A0, A1 = _edge_pass(y7, ei_pp2, ea_pp2, N_P2, "conv2")
    h7 = _tc(lambda a, b, A, B, s, wz: jnp.maximum(a + b + dot(A + B, wz) + s, 0.),
             N_P2, [32], [q0, q1, A0, A1, s7], [_wz(Wm7[48:52], bm7, 32)])

    q0, q1 = _edge_pass(h7, ei_unpool1, ea_unpool1, N_P1, "pool")
    y8, s8 = _tc(lambda a, b, c_, wm1, wm2_, ws1, ws2_, bs_:
                 ((lambda u: (dot(u, wm1) + dot(c_, wm2_),
                              dot(u, ws1) + dot(c_, ws2_) + bs_))(a + b)),
                 N_P1, [16, 16], [q0, q1, c2],
                 [Wm8[:32], Wm8[32:48], Ws8[:32], Ws8[32:48], bs8.reshape(1, -1)])
    z8 = lambda ep: _z_edges(ea_pp1, Wm8[48:52], bm8, 16, ep)
    q0, q1 = _edge_pass(y8, ei_pp1, z8, N_P1, "conv")
    h8 = _tc(lambda a, b, s: jnp.maximum(a + b + s, 0.), N_P1, [16], [q0, q1, s8], [])

    q0, q1 = _edge_pass(h8, ei_unpool0, ea_unpool0, N_P0, "pool")
    y9, s9 = _tc(lambda a, b, c_, wm1, wm2_, ws1, ws2_, bs_:
                 ((lambda u: (dot(u, wm1) + dot(c_, wm2_),
                              dot(u, ws1) + dot(c_, ws2_) + bs_))(a + b)),
                 N_P0, [16, 16], [q0, q1, c1],
                 [Wm9[:16], Wm9[16:32], Ws9[:16], Ws9[16:32], bs9.reshape(1, -1)])
    z9 = lambda ep: _z_edges(ea_pp0, Wm9[32:36], bm9, 16, ep)
    q0, q1 = _edge_pass(y9, ei_pp0, z9, N_P0, "conv")

    y9b = _tc(lambda a, b, s, w: dot(jnp.maximum(a + b + s, 0.), w),
              N_P0, [16], [q0, q1, s9], [W9b[:16]])
    z9b = lambda ep: _z_edges(ea_pc, W9b[16:20], b9b, 16, ep)
    q0, q1 = _edge_pass(y9b, ei_pc, z9b, N_C, "conv")
    out = _tc(lambda a, b, wf, bf_: dot(jnp.maximum(a + b, 0.), wf) + bf_,
              N_C, [1], [q0, q1], [Wf, bf.reshape(1, -1)])
    return out
